# baseline (speedup 1.0000x reference)
import jax
import jax.numpy as jnp
from jax.experimental import pallas as pl

NLAYER = 3
NGRAPHS = 128


def kernel(x, s, W_pre, b_pre, W_es, b_es, W1s, b1s, W2s, b2s, Wgs, bgs, W_hp, b_hp, W_post, b_post, W_ro, b_ro, edge_index, batch):
    src = edge_index[0]
    dst = edge_index[1]
    N = x.shape[0]
    x = x @ W_pre + b_pre
    s = s @ W_es + b_es
    deg = jnp.zeros((N,), jnp.float32).at[dst].add(1.0) + 1.0
    dinv = 1.0 / jnp.sqrt(deg)
    norm = dinv[src] * dinv[dst]
    self_norm = (dinv * dinv)[:, None]
    for i in range(NLAYER):
        h = jnp.concatenate([x, s], axis=-1)
        agg = jax.ops.segment_sum(h[src], dst, num_segments=N)
        h = h + agg
        h = jnp.maximum(h @ W1s[i] + b1s[i], 0.0) @ W2s[i] + b2s[i]
        x = jax.nn.relu(h)
        st = s @ Wgs[i]
        gout = jax.ops.segment_sum(st[src] * norm[:, None], dst, num_segments=N) + st * self_norm + bgs[i]
        s = jnp.tanh(gout)
    x = jnp.concatenate([x, s], axis=-1) @ W_hp + b_hp
    pooled = jax.ops.segment_sum(x, batch, num_segments=NGRAPHS)
    x = jax.nn.relu(pooled @ W_post + b_post)
    x = x @ W_ro + b_ro
    return jax.nn.log_softmax(x, axis=1)


# speedup vs baseline: 3.8161x; 3.8161x over previous
import jax
import jax.numpy as jnp
from jax import lax
from jax.experimental import pallas as pl
from jax.experimental.pallas import tpu as pltpu
from jax.experimental.pallas import tpu_sc as plsc

N = 10000
NP = 10240
E = 320000
NT = 16
CH = 128
NCHUNK = 160
EP = NT * NCHUNK * CH
HN = NP // 2
AR = HN + 256
RPT = AR // NT
W = 384
NGR = 128
NCLS = 10
F32 = jnp.float32

_MESH = dict(core_axis_name="c", subcore_axis_name="s", num_cores=2,
             num_subcores=16)


def _remap_dst(dst_v, c):
    base = c * HN

    def step(t, carry):
        j = t // (CH // 16)
        k = t % (CH // 16)
        v = dst_v[j, pl.ds(k * 16, 16)]
        loc = v - base
        ok = jnp.logical_and(loc >= 0, loc < HN)
        dst_v[j, pl.ds(k * 16, 16)] = jnp.where(ok, loc, HN + (v & 255))
        return carry

    lax.fori_loop(0, NCHUNK * (CH // 16), step, 0, unroll=8)


def _deg_body(dst_hbm, out_hbm, dst_v, ones_v, zbuf, acc1):
    c = lax.axis_index("c")
    sid = lax.axis_index("s")
    pltpu.sync_copy(dst_hbm.at[sid], dst_v)
    for i in range(CH // 16):
        ones_v[pl.ds(i * 16, 16)] = jnp.full((16,), 1.0, F32)
    for i in range(RPT // 16):
        zbuf[pl.ds(i * 16, 16)] = jnp.zeros((16,), F32)
    _remap_dst(dst_v, c)
    pltpu.sync_copy(zbuf, acc1.at[pl.ds(sid * RPT, RPT)])
    plsc.subcore_barrier()

    def chunk(j, carry):
        pltpu.sync_copy(ones_v, acc1.at[dst_v.at[j]], add=True)
        return carry

    lax.fori_loop(0, NCHUNK, chunk, 0)
    plsc.subcore_barrier()

    @pl.when(sid < AR // 384)
    def _():
        off = pl.multiple_of(c * AR + sid * 384, 128)
        pltpu.sync_copy(acc1.at[pl.ds(sid * 384, 384)],
                        out_hbm.at[pl.ds(off, 384)])


def _sc_deg(dst_r):
    k = pl.kernel(
        _deg_body,
        out_type=jax.ShapeDtypeStruct((2 * AR,), F32),
        mesh=plsc.VectorSubcoreMesh(**_MESH),
        scratch_types=[
            pltpu.VMEM((NCHUNK, CH), jnp.int32),
            pltpu.VMEM((CH,), F32),
            pltpu.VMEM((RPT,), F32),
            pltpu.VMEM_SHARED((AR,), F32),
        ],
    )
    return k(dst_r)


def _agg_body(t0_hbm, t1_hbm, t2_hbm, src_hbm, dst_hbm, z_hbm, out_hbm,
              src_v, dst_v, rows_v, sem, acc):
    c = lax.axis_index("c")
    sid = lax.axis_index("s")
    pltpu.sync_copy(src_hbm.at[sid], src_v)
    pltpu.sync_copy(dst_hbm.at[sid], dst_v)
    _remap_dst(dst_v, c)
    for k, tab in enumerate((t0_hbm, t1_hbm, t2_hbm)):
        pltpu.sync_copy(z_hbm, acc.at[pl.ds(sid * RPT, RPT)])
        plsc.subcore_barrier()

        def chunk(j, carry, tab=tab):
            pltpu.async_copy(tab.at[src_v.at[j]], rows_v, sem).wait()
            pltpu.sync_copy(rows_v, acc.at[dst_v.at[j]], add=True)
            return carry

        lax.fori_loop(0, NCHUNK, chunk, 0)
        plsc.subcore_barrier()
        pltpu.sync_copy(acc.at[pl.ds(sid * RPT, RPT)],
                        out_hbm.at[c, pl.ds(sid * RPT, RPT),
                                   pl.ds(k * 128, 128)])


def _sc_agg(t0, t1, t2, src_r, dst_r, zeros_t):
    k = pl.kernel(
        _agg_body,
        out_type=jax.ShapeDtypeStruct((2, AR, W), F32),
        mesh=plsc.VectorSubcoreMesh(**_MESH),
        scratch_types=[
            pltpu.VMEM((NCHUNK, CH), jnp.int32),
            pltpu.VMEM((NCHUNK, CH), jnp.int32),
            pltpu.VMEM((CH, 128), F32),
            pltpu.SemaphoreType.DMA,
            pltpu.VMEM_SHARED((AR, 128), F32),
        ],
    )
    return k(t0, t1, t2, src_r, dst_r, zeros_t)


_BLK = 1280
_GRID = NP // _BLK


def _pre_body(xp_ref, sp_ref, wpre, bpre, wes, bes, wg, dg,
              x1_ref, s1_ref, t2_ref):
    xv = xp_ref[...] @ wpre[...] + bpre[...]
    sv = sp_ref[...] @ wes[...] + bes[...]
    dinv = lax.rsqrt(dg[...] + 1.0)
    st = sv @ wg[...]
    x1_ref[...] = xv
    s1_ref[...] = sv
    t2_ref[...] = st * dinv


def _tc_pre(xp, sp, wpre, bpre, wes, bes, wg, dg):
    row = lambda i: (i, 0)
    whole = lambda i: (0, 0)
    return pl.pallas_call(
        _pre_body,
        grid=(_GRID,),
        in_specs=[
            pl.BlockSpec((_BLK, 128), row),
            pl.BlockSpec((_BLK, 16), row),
            pl.BlockSpec((128, 128), whole),
            pl.BlockSpec((1, 128), whole),
            pl.BlockSpec((16, 128), whole),
            pl.BlockSpec((1, 128), whole),
            pl.BlockSpec((128, 128), whole),
            pl.BlockSpec((_BLK, 1), row),
        ],
        out_specs=[
            pl.BlockSpec((_BLK, 128), row),
            pl.BlockSpec((_BLK, 128), row),
            pl.BlockSpec((_BLK, 128), row),
        ],
        out_shape=[
            jax.ShapeDtypeStruct((NP, 128), F32),
            jax.ShapeDtypeStruct((NP, 128), F32),
            jax.ShapeDtypeStruct((NP, 128), F32),
        ],
    )(xp, sp, wpre, bpre, wes, bes, wg, dg)


def _layer_update(x, s, t2prev, a, dinv, w1, b1, w2, b2, bg):
    hx = x + a[:, :128]
    hs = s + a[:, 128:256]
    h = jnp.concatenate([hx, hs], axis=1)
    m = jnp.maximum(h @ w1 + b1, 0.0) @ w2 + b2
    xn = jnp.maximum(m, 0.0)
    sn = jnp.tanh(dinv * (a[:, 256:] + t2prev) + bg)
    return xn, sn


def _lay_body(x_ref, s_ref, tp_ref, a_ref, dg, w1, b1, w2, b2, bg, wgn,
              xo_ref, so_ref, to_ref):
    dinv = lax.rsqrt(dg[...] + 1.0)
    xn, sn = _layer_update(x_ref[...], s_ref[...], tp_ref[...], a_ref[...],
                           dinv, w1[...], b1[...], w2[...], b2[...], bg[...])
    stn = sn @ wgn[...]
    xo_ref[...] = xn
    so_ref[...] = sn
    to_ref[...] = stn * dinv


def _tc_layer(x, s, tp, a, dg, w1, b1, w2, b2, bg, wgn):
    row = lambda i: (i, 0)
    whole = lambda i: (0, 0)
    return pl.pallas_call(
        _lay_body,
        grid=(_GRID,),
        in_specs=[
            pl.BlockSpec((_BLK, 128), row),
            pl.BlockSpec((_BLK, 128), row),
            pl.BlockSpec((_BLK, 128), row),
            pl.BlockSpec((_BLK, W), row),
            pl.BlockSpec((_BLK, 1), row),
            pl.BlockSpec((256, 128), whole),
            pl.BlockSpec((1, 128), whole),
            pl.BlockSpec((128, 128), whole),
            pl.BlockSpec((1, 128), whole),
            pl.BlockSpec((1, 128), whole),
            pl.BlockSpec((128, 128), whole),
        ],
        out_specs=[
            pl.BlockSpec((_BLK, 128), row),
            pl.BlockSpec((_BLK, 128), row),
            pl.BlockSpec((_BLK, 128), row),
        ],
        out_shape=[
            jax.ShapeDtypeStruct((NP, 128), F32),
            jax.ShapeDtypeStruct((NP, 128), F32),
            jax.ShapeDtypeStruct((NP, 128), F32),
        ],
    )(x, s, tp, a, dg, w1, b1, w2, b2, bg, wgn)


def _fin_body(x_ref, s_ref, tp_ref, a_ref, dg, w1, b1, w2, b2, bg,
              whp, bhp, bat_ref, wpost, bpost, wro, bro,
              out_ref, pool_acc):
    i = pl.program_id(0)
    dinv = lax.rsqrt(dg[...] + 1.0)
    xn, sn = _layer_update(x_ref[...], s_ref[...], tp_ref[...], a_ref[...],
                           dinv, w1[...], b1[...], w2[...], b2[...], bg[...])
    hp = jnp.concatenate([xn, sn], axis=1) @ whp[...] + bhp[...]
    onehot = (lax.broadcasted_iota(jnp.int32, (NGR, _BLK), 0)
              == bat_ref[0]).astype(F32)
    part = onehot @ hp

    @pl.when(i == 0)
    def _():
        pool_acc[...] = part

    @pl.when(i > 0)
    def _():
        pool_acc[...] = pool_acc[...] + part

    @pl.when(i == _GRID - 1)
    def _():
        p2 = jnp.maximum(pool_acc[...] @ wpost[...] + bpost[...], 0.0)
        logits = p2 @ wro[...] + bro[...]
        m = jnp.max(logits, axis=1, keepdims=True)
        z = logits - m
        lse = jnp.log(jnp.sum(jnp.exp(z), axis=1, keepdims=True))
        out_ref[...] = z - lse


def _tc_final(x, s, tp, a, dg, w1, b1, w2, b2, bg,
              whp, bhp, batr, wpost, bpost, wro, bro):
    row = lambda i: (i, 0)
    whole = lambda i: (0, 0)
    return pl.pallas_call(
        _fin_body,
        grid=(_GRID,),
        in_specs=[
            pl.BlockSpec((_BLK, 128), row),
            pl.BlockSpec((_BLK, 128), row),
            pl.BlockSpec((_BLK, 128), row),
            pl.BlockSpec((_BLK, W), row),
            pl.BlockSpec((_BLK, 1), row),
            pl.BlockSpec((256, 128), whole),
            pl.BlockSpec((1, 128), whole),
            pl.BlockSpec((128, 128), whole),
            pl.BlockSpec((1, 128), whole),
            pl.BlockSpec((1, 128), whole),
            pl.BlockSpec((256, 128), whole),
            pl.BlockSpec((1, 128), whole),
            pl.BlockSpec((1, 1, _BLK), lambda i: (i, 0, 0)),
            pl.BlockSpec((128, 128), whole),
            pl.BlockSpec((1, 128), whole),
            pl.BlockSpec((128, NCLS), whole),
            pl.BlockSpec((1, NCLS), whole),
        ],
        out_specs=pl.BlockSpec((NGR, NCLS), whole),
        out_shape=jax.ShapeDtypeStruct((NGR, NCLS), F32),
        scratch_shapes=[pltpu.VMEM((NGR, 128), F32)],
    )(x, s, tp, a, dg, w1, b1, w2, b2, bg,
      whp, bhp, batr, wpost, bpost, wro, bro)


def kernel(x, s, W_pre, b_pre, W_es, b_es, W1s, b1s, W2s, b2s, Wgs, bgs,
           W_hp, b_hp, W_post, b_post, W_ro, b_ro, edge_index, batch):
    src = edge_index[0]
    dst = edge_index[1]
    npad = EP - E
    pad_src = jnp.arange(npad, dtype=jnp.int32) % 2048
    pad_dst = N + jnp.arange(npad, dtype=jnp.int32) % (NP - N)
    src_r = jnp.concatenate([src, pad_src]).reshape(NT, NCHUNK, CH)
    dst_r = jnp.concatenate([dst, pad_dst]).reshape(NT, NCHUNK, CH)
    zeros_t = jnp.zeros((RPT, 128), F32)

    xp = jnp.pad(x, ((0, NP - N), (0, 0)))
    sp = jnp.pad(s, ((0, NP - N), (0, 0)))
    batr = jnp.pad(batch, (0, NP - N),
                   constant_values=NGR).reshape(_GRID, 1, _BLK)

    degp = _sc_deg(dst_r).reshape(2, AR)
    dg = jnp.concatenate([degp[0, :HN], degp[1, :HN]])[:, None]

    r1 = lambda a: a.reshape(1, -1)
    xc, sc, t2 = _tc_pre(xp, sp, W_pre, r1(b_pre), W_es, r1(b_es),
                         Wgs[0], dg)
    for i in range(3):
        aggp = _sc_agg(xc, sc, t2, src_r, dst_r, zeros_t)
        agg = jnp.concatenate([aggp[0, :HN], aggp[1, :HN]], axis=0)
        if i < 2:
            xc, sc, t2 = _tc_layer(
                xc, sc, t2, agg, dg,
                W1s[i], r1(b1s[i]), W2s[i], r1(b2s[i]), r1(bgs[i]),
                Wgs[i + 1])
        else:
            out = _tc_final(
                xc, sc, t2, agg, dg,
                W1s[i], r1(b1s[i]), W2s[i], r1(b2s[i]), r1(bgs[i]),
                W_hp, r1(b_hp), batr, W_post, r1(b_post), W_ro, r1(b_ro))
    return out


# speedup vs baseline: 5.2031x; 1.3634x over previous
import jax
import jax.numpy as jnp
from jax import lax
from jax.experimental import pallas as pl
from jax.experimental.pallas import tpu as pltpu
from jax.experimental.pallas import tpu_sc as plsc

N = 10000
NP = 10240
E = 320000
NT = 16
CH = 128
NCHUNK = 160
EP = NT * NCHUNK * CH
HN = NP // 2
AR = HN + 256
RPT = AR // NT
W = 384
NGR = 128
NCLS = 10
F32 = jnp.float32

_MESH = dict(core_axis_name="c", subcore_axis_name="s", num_cores=2,
             num_subcores=16)


def _remap_dst(dst_v, c):
    base = c * HN

    def step(t, carry):
        j = t // (CH // 16)
        k = t % (CH // 16)
        v = dst_v[j, pl.ds(k * 16, 16)]
        loc = v - base
        ok = jnp.logical_and(loc >= 0, loc < HN)
        dst_v[j, pl.ds(k * 16, 16)] = jnp.where(ok, loc, HN + (v & 255))
        return carry

    lax.fori_loop(0, NCHUNK * (CH // 16), step, 0, unroll=8)


def _deg_body(dst_hbm, out_hbm, dst_v, ones_v, zbuf, acc1):
    c = lax.axis_index("c")
    sid = lax.axis_index("s")
    pltpu.sync_copy(dst_hbm.at[sid], dst_v)
    for i in range(CH // 16):
        ones_v[pl.ds(i * 16, 16)] = jnp.full((16,), 1.0, F32)
    for i in range(RPT // 16):
        zbuf[pl.ds(i * 16, 16)] = jnp.zeros((16,), F32)
    _remap_dst(dst_v, c)
    pltpu.sync_copy(zbuf, acc1.at[pl.ds(sid * RPT, RPT)])
    plsc.subcore_barrier()

    def chunk(j, carry):
        pltpu.sync_copy(ones_v, acc1.at[dst_v.at[j]], add=True)
        return carry

    lax.fori_loop(0, NCHUNK, chunk, 0)
    plsc.subcore_barrier()

    @pl.when(sid < AR // 384)
    def _():
        off = pl.multiple_of(c * AR + sid * 384, 128)
        pltpu.sync_copy(acc1.at[pl.ds(sid * 384, 384)],
                        out_hbm.at[pl.ds(off, 384)])


def _sc_deg(dst_r):
    k = pl.kernel(
        _deg_body,
        out_type=jax.ShapeDtypeStruct((2 * AR,), F32),
        mesh=plsc.VectorSubcoreMesh(**_MESH),
        scratch_types=[
            pltpu.VMEM((NCHUNK, CH), jnp.int32),
            pltpu.VMEM((CH,), F32),
            pltpu.VMEM((RPT,), F32),
            pltpu.VMEM_SHARED((AR,), F32),
        ],
    )
    return k(dst_r)


def _agg_body(t0_hbm, t1_hbm, t2_hbm, src_hbm, dst_hbm, z_hbm, out_hbm,
              src_v, dst_v, rows_v, rows_b, sem, sem_b, acc):
    c = lax.axis_index("c")
    sid = lax.axis_index("s")
    pltpu.sync_copy(src_hbm.at[sid], src_v)
    pltpu.sync_copy(dst_hbm.at[sid], dst_v)
    _remap_dst(dst_v, c)
    for k, tab in enumerate((t0_hbm, t1_hbm, t2_hbm)):
        pltpu.sync_copy(z_hbm, acc.at[pl.ds(sid * RPT, RPT)])
        plsc.subcore_barrier()

        def wait_rows(buf, sem_, tab=tab):
            pltpu.make_async_copy(tab.at[src_v.at[0]], buf, sem_).wait()

        pltpu.async_copy(tab.at[src_v.at[0]], rows_v, sem)

        def pair(t, carry, tab=tab):
            j = t * 2
            wait_rows(rows_v, sem)
            pltpu.async_copy(tab.at[src_v.at[j + 1]], rows_b, sem_b)
            pltpu.sync_copy(rows_v, acc.at[dst_v.at[j]], add=True)
            wait_rows(rows_b, sem_b)

            @pl.when(t < NCHUNK // 2 - 1)
            def _():
                pltpu.async_copy(tab.at[src_v.at[j + 2]], rows_v, sem)

            pltpu.sync_copy(rows_b, acc.at[dst_v.at[j + 1]], add=True)
            return carry

        lax.fori_loop(0, NCHUNK // 2, pair, 0)
        plsc.subcore_barrier()
        pltpu.sync_copy(acc.at[pl.ds(sid * RPT, RPT)],
                        out_hbm.at[c, pl.ds(sid * RPT, RPT),
                                   pl.ds(k * 128, 128)])


def _sc_agg(t0, t1, t2, src_r, dst_r, zeros_t):
    k = pl.kernel(
        _agg_body,
        out_type=jax.ShapeDtypeStruct((2, AR, W), F32),
        mesh=plsc.VectorSubcoreMesh(**_MESH),
        scratch_types=[
            pltpu.VMEM((NCHUNK, CH), jnp.int32),
            pltpu.VMEM((NCHUNK, CH), jnp.int32),
            pltpu.VMEM((CH, 128), F32),
            pltpu.VMEM((CH, 128), F32),
            pltpu.SemaphoreType.DMA,
            pltpu.SemaphoreType.DMA,
            pltpu.VMEM_SHARED((AR, 128), F32),
        ],
    )
    return k(t0, t1, t2, src_r, dst_r, zeros_t)


_BLK = 1280
_GRID = NP // _BLK


def _pre_body(xp_ref, sp_ref, wpre, bpre, wes, bes, wg, dg,
              x1_ref, s1_ref, t2_ref):
    xv = xp_ref[...] @ wpre[...] + bpre[...]
    sv = sp_ref[...] @ wes[...] + bes[...]
    dinv = lax.rsqrt(dg[...] + 1.0)
    st = sv @ wg[...]
    x1_ref[...] = xv
    s1_ref[...] = sv
    t2_ref[...] = st * dinv


def _tc_pre(xp, sp, wpre, bpre, wes, bes, wg, dg):
    row = lambda i: (i, 0)
    whole = lambda i: (0, 0)
    return pl.pallas_call(
        _pre_body,
        grid=(_GRID,),
        in_specs=[
            pl.BlockSpec((_BLK, 128), row),
            pl.BlockSpec((_BLK, 16), row),
            pl.BlockSpec((128, 128), whole),
            pl.BlockSpec((1, 128), whole),
            pl.BlockSpec((16, 128), whole),
            pl.BlockSpec((1, 128), whole),
            pl.BlockSpec((128, 128), whole),
            pl.BlockSpec((_BLK, 1), row),
        ],
        out_specs=[
            pl.BlockSpec((_BLK, 128), row),
            pl.BlockSpec((_BLK, 128), row),
            pl.BlockSpec((_BLK, 128), row),
        ],
        out_shape=[
            jax.ShapeDtypeStruct((NP, 128), F32),
            jax.ShapeDtypeStruct((NP, 128), F32),
            jax.ShapeDtypeStruct((NP, 128), F32),
        ],
    )(xp, sp, wpre, bpre, wes, bes, wg, dg)


def _layer_update(x, s, t2prev, a, dinv, w1, b1, w2, b2, bg):
    hx = x + a[:, :128]
    hs = s + a[:, 128:256]
    h = jnp.concatenate([hx, hs], axis=1)
    m = jnp.maximum(h @ w1 + b1, 0.0) @ w2 + b2
    xn = jnp.maximum(m, 0.0)
    sn = jnp.tanh(dinv * (a[:, 256:] + t2prev) + bg)
    return xn, sn


def _lay_body(x_ref, s_ref, tp_ref, a_ref, dg, w1, b1, w2, b2, bg, wgn,
              xo_ref, so_ref, to_ref):
    dinv = lax.rsqrt(dg[...] + 1.0)
    xn, sn = _layer_update(x_ref[...], s_ref[...], tp_ref[...], a_ref[...],
                           dinv, w1[...], b1[...], w2[...], b2[...], bg[...])
    stn = sn @ wgn[...]
    xo_ref[...] = xn
    so_ref[...] = sn
    to_ref[...] = stn * dinv


def _tc_layer(x, s, tp, a, dg, w1, b1, w2, b2, bg, wgn):
    row = lambda i: (i, 0)
    whole = lambda i: (0, 0)
    return pl.pallas_call(
        _lay_body,
        grid=(_GRID,),
        in_specs=[
            pl.BlockSpec((_BLK, 128), row),
            pl.BlockSpec((_BLK, 128), row),
            pl.BlockSpec((_BLK, 128), row),
            pl.BlockSpec((_BLK, W), row),
            pl.BlockSpec((_BLK, 1), row),
            pl.BlockSpec((256, 128), whole),
            pl.BlockSpec((1, 128), whole),
            pl.BlockSpec((128, 128), whole),
            pl.BlockSpec((1, 128), whole),
            pl.BlockSpec((1, 128), whole),
            pl.BlockSpec((128, 128), whole),
        ],
        out_specs=[
            pl.BlockSpec((_BLK, 128), row),
            pl.BlockSpec((_BLK, 128), row),
            pl.BlockSpec((_BLK, 128), row),
        ],
        out_shape=[
            jax.ShapeDtypeStruct((NP, 128), F32),
            jax.ShapeDtypeStruct((NP, 128), F32),
            jax.ShapeDtypeStruct((NP, 128), F32),
        ],
    )(x, s, tp, a, dg, w1, b1, w2, b2, bg, wgn)


def _fin_body(x_ref, s_ref, tp_ref, a_ref, dg, w1, b1, w2, b2, bg,
              whp, bhp, bat_ref, wpost, bpost, wro, bro,
              out_ref, pool_acc):
    i = pl.program_id(0)
    dinv = lax.rsqrt(dg[...] + 1.0)
    xn, sn = _layer_update(x_ref[...], s_ref[...], tp_ref[...], a_ref[...],
                           dinv, w1[...], b1[...], w2[...], b2[...], bg[...])
    hp = jnp.concatenate([xn, sn], axis=1) @ whp[...] + bhp[...]
    onehot = (lax.broadcasted_iota(jnp.int32, (NGR, _BLK), 0)
              == bat_ref[0]).astype(F32)
    part = onehot @ hp

    @pl.when(i == 0)
    def _():
        pool_acc[...] = part

    @pl.when(i > 0)
    def _():
        pool_acc[...] = pool_acc[...] + part

    @pl.when(i == _GRID - 1)
    def _():
        p2 = jnp.maximum(pool_acc[...] @ wpost[...] + bpost[...], 0.0)
        logits = p2 @ wro[...] + bro[...]
        m = jnp.max(logits, axis=1, keepdims=True)
        z = logits - m
        lse = jnp.log(jnp.sum(jnp.exp(z), axis=1, keepdims=True))
        out_ref[...] = z - lse


def _tc_final(x, s, tp, a, dg, w1, b1, w2, b2, bg,
              whp, bhp, batr, wpost, bpost, wro, bro):
    row = lambda i: (i, 0)
    whole = lambda i: (0, 0)
    return pl.pallas_call(
        _fin_body,
        grid=(_GRID,),
        in_specs=[
            pl.BlockSpec((_BLK, 128), row),
            pl.BlockSpec((_BLK, 128), row),
            pl.BlockSpec((_BLK, 128), row),
            pl.BlockSpec((_BLK, W), row),
            pl.BlockSpec((_BLK, 1), row),
            pl.BlockSpec((256, 128), whole),
            pl.BlockSpec((1, 128), whole),
            pl.BlockSpec((128, 128), whole),
            pl.BlockSpec((1, 128), whole),
            pl.BlockSpec((1, 128), whole),
            pl.BlockSpec((256, 128), whole),
            pl.BlockSpec((1, 128), whole),
            pl.BlockSpec((1, 1, _BLK), lambda i: (i, 0, 0)),
            pl.BlockSpec((128, 128), whole),
            pl.BlockSpec((1, 128), whole),
            pl.BlockSpec((128, NCLS), whole),
            pl.BlockSpec((1, NCLS), whole),
        ],
        out_specs=pl.BlockSpec((NGR, NCLS), whole),
        out_shape=jax.ShapeDtypeStruct((NGR, NCLS), F32),
        scratch_shapes=[pltpu.VMEM((NGR, 128), F32)],
    )(x, s, tp, a, dg, w1, b1, w2, b2, bg,
      whp, bhp, batr, wpost, bpost, wro, bro)


def kernel(x, s, W_pre, b_pre, W_es, b_es, W1s, b1s, W2s, b2s, Wgs, bgs,
           W_hp, b_hp, W_post, b_post, W_ro, b_ro, edge_index, batch):
    src = edge_index[0]
    dst = edge_index[1]
    npad = EP - E
    pad_src = jnp.arange(npad, dtype=jnp.int32) % 2048
    pad_dst = N + jnp.arange(npad, dtype=jnp.int32) % (NP - N)
    src_r = jnp.concatenate([src, pad_src]).reshape(NT, NCHUNK, CH)
    dst_r = jnp.concatenate([dst, pad_dst]).reshape(NT, NCHUNK, CH)
    zeros_t = jnp.zeros((RPT, 128), F32)

    xp = jnp.pad(x, ((0, NP - N), (0, 0)))
    sp = jnp.pad(s, ((0, NP - N), (0, 0)))
    batr = jnp.pad(batch, (0, NP - N),
                   constant_values=NGR).reshape(_GRID, 1, _BLK)

    degp = _sc_deg(dst_r).reshape(2, AR)
    dg = jnp.concatenate([degp[0, :HN], degp[1, :HN]])[:, None]

    r1 = lambda a: a.reshape(1, -1)
    xc, sc, t2 = _tc_pre(xp, sp, W_pre, r1(b_pre), W_es, r1(b_es),
                         Wgs[0], dg)
    for i in range(3):
        aggp = _sc_agg(xc, sc, t2, src_r, dst_r, zeros_t)
        agg = jnp.concatenate([aggp[0, :HN], aggp[1, :HN]], axis=0)
        if i < 2:
            xc, sc, t2 = _tc_layer(
                xc, sc, t2, agg, dg,
                W1s[i], r1(b1s[i]), W2s[i], r1(b2s[i]), r1(bgs[i]),
                Wgs[i + 1])
        else:
            out = _tc_final(
                xc, sc, t2, agg, dg,
                W1s[i], r1(b1s[i]), W2s[i], r1(b2s[i]), r1(bgs[i]),
                W_hp, r1(b_hp), batr, W_post, r1(b_post), W_ro, r1(b_ro))
    return out


# speedup vs baseline: 6.6392x; 1.2760x over previous
import jax
import jax.numpy as jnp
from jax import lax
from jax.experimental import pallas as pl
from jax.experimental.pallas import tpu as pltpu
from jax.experimental.pallas import tpu_sc as plsc

N = 10000
NP = 10240
E = 320000
NT = 16
CH = 64
NCHUNK = 160
EP = 2 * NT * NCHUNK * CH
RPT = NP // NT
W = 384
NGR = 128
NCLS = 10
F32 = jnp.float32

_MESH = dict(core_axis_name="c", subcore_axis_name="s", num_cores=2,
             num_subcores=16)


def _deg_body(dst_hbm, out_hbm, dst_v, ones_v, zbuf, acc1):
    c = lax.axis_index("c")
    sid = lax.axis_index("s")
    pltpu.sync_copy(dst_hbm.at[c, sid], dst_v)
    for i in range(CH // 16):
        ones_v[pl.ds(i * 16, 16)] = jnp.full((16,), 1.0, F32)
    for i in range(RPT // 16):
        zbuf[pl.ds(i * 16, 16)] = jnp.zeros((16,), F32)
    pltpu.sync_copy(zbuf, acc1.at[pl.ds(sid * RPT, RPT)])
    plsc.subcore_barrier()

    def chunk(j, carry):
        pltpu.sync_copy(ones_v, acc1.at[dst_v.at[j]], add=True)
        return carry

    lax.fori_loop(0, NCHUNK, chunk, 0)
    plsc.subcore_barrier()
    off = pl.multiple_of(c * NP + sid * RPT, 128)
    pltpu.sync_copy(acc1.at[pl.ds(sid * RPT, RPT)],
                    out_hbm.at[pl.ds(off, RPT)])


def _sc_deg(dst_r):
    k = pl.kernel(
        _deg_body,
        out_type=jax.ShapeDtypeStruct((2 * NP,), F32),
        mesh=plsc.VectorSubcoreMesh(**_MESH),
        scratch_types=[
            pltpu.VMEM((NCHUNK, CH), jnp.int32),
            pltpu.VMEM((CH,), F32),
            pltpu.VMEM((RPT,), F32),
            pltpu.VMEM_SHARED((NP,), F32),
        ],
    )
    return k(dst_r)


HC = NCHUNK // 2


def _agg_body(t0_hbm, t1_hbm, t2_hbm, src_hbm, dst_hbm, z_hbm, out_hbm,
              src_v, dst_v, rows_v, rows_b, sem, sem_b, acc):
    c = lax.axis_index("c")
    sid = lax.axis_index("s")
    for k, tab in enumerate((t0_hbm, t1_hbm, t2_hbm)):
        pltpu.sync_copy(z_hbm, acc.at[pl.ds(sid * RPT, RPT)])
        plsc.subcore_barrier()

        def wait_rows(buf, sem_, tab=tab):
            pltpu.make_async_copy(tab.at[src_v.at[0]], buf, sem_).wait()

        for half in range(2):
            pltpu.sync_copy(src_hbm.at[c, sid, pl.ds(half * HC, HC)], src_v)
            pltpu.sync_copy(dst_hbm.at[c, sid, pl.ds(half * HC, HC)], dst_v)

            pltpu.async_copy(tab.at[src_v.at[0]], rows_v, sem)

            def pair(t, carry, tab=tab):
                j = t * 2
                wait_rows(rows_v, sem)
                pltpu.async_copy(tab.at[src_v.at[j + 1]], rows_b, sem_b)
                pltpu.sync_copy(rows_v, acc.at[dst_v.at[j]], add=True)
                wait_rows(rows_b, sem_b)

                @pl.when(t < HC // 2 - 1)
                def _():
                    pltpu.async_copy(tab.at[src_v.at[j + 2]], rows_v, sem)

                pltpu.sync_copy(rows_b, acc.at[dst_v.at[j + 1]], add=True)
                return carry

            lax.fori_loop(0, HC // 2, pair, 0)
        plsc.subcore_barrier()
        pltpu.sync_copy(acc.at[pl.ds(sid * RPT, RPT)],
                        out_hbm.at[c, pl.ds(sid * RPT, RPT),
                                   pl.ds(k * 128, 128)])


def _sc_agg(t0, t1, t2, src_r, dst_r, zeros_t):
    k = pl.kernel(
        _agg_body,
        out_type=jax.ShapeDtypeStruct((2, NP, W), F32),
        mesh=plsc.VectorSubcoreMesh(**_MESH),
        scratch_types=[
            pltpu.VMEM((HC, CH), jnp.int32),
            pltpu.VMEM((HC, CH), jnp.int32),
            pltpu.VMEM((CH, 128), F32),
            pltpu.VMEM((CH, 128), F32),
            pltpu.SemaphoreType.DMA,
            pltpu.SemaphoreType.DMA,
            pltpu.VMEM_SHARED((NP, 128), F32),
        ],
    )
    return k(t0, t1, t2, src_r, dst_r, zeros_t)


_BLK = 1280
_GRID = NP // _BLK


def _pre_body(xp_ref, sp_ref, wpre, bpre, wes, bes, wg, dg,
              x1_ref, s1_ref, t2_ref):
    xv = xp_ref[...] @ wpre[...] + bpre[...]
    sv = sp_ref[...] @ wes[...] + bes[...]
    dinv = lax.rsqrt(dg[...] + 1.0)
    st = sv @ wg[...]
    x1_ref[...] = xv
    s1_ref[...] = sv
    t2_ref[...] = st * dinv


def _tc_pre(xp, sp, wpre, bpre, wes, bes, wg, dg):
    row = lambda i: (i, 0)
    whole = lambda i: (0, 0)
    return pl.pallas_call(
        _pre_body,
        grid=(_GRID,),
        in_specs=[
            pl.BlockSpec((_BLK, 128), row),
            pl.BlockSpec((_BLK, 16), row),
            pl.BlockSpec((128, 128), whole),
            pl.BlockSpec((1, 128), whole),
            pl.BlockSpec((16, 128), whole),
            pl.BlockSpec((1, 128), whole),
            pl.BlockSpec((128, 128), whole),
            pl.BlockSpec((_BLK, 1), row),
        ],
        out_specs=[
            pl.BlockSpec((_BLK, 128), row),
            pl.BlockSpec((_BLK, 128), row),
            pl.BlockSpec((_BLK, 128), row),
        ],
        out_shape=[
            jax.ShapeDtypeStruct((NP, 128), F32),
            jax.ShapeDtypeStruct((NP, 128), F32),
            jax.ShapeDtypeStruct((NP, 128), F32),
        ],
    )(xp, sp, wpre, bpre, wes, bes, wg, dg)


def _layer_update(x, s, t2prev, a, dinv, w1, b1, w2, b2, bg):
    hx = x + a[:, :128]
    hs = s + a[:, 128:256]
    h = jnp.concatenate([hx, hs], axis=1)
    m = jnp.maximum(h @ w1 + b1, 0.0) @ w2 + b2
    xn = jnp.maximum(m, 0.0)
    sn = jnp.tanh(dinv * (a[:, 256:] + t2prev) + bg)
    return xn, sn


def _lay_body(x_ref, s_ref, tp_ref, a0_ref, a1_ref, dg, w1, b1, w2, b2, bg,
              wgn, xo_ref, so_ref, to_ref):
    dinv = lax.rsqrt(dg[...] + 1.0)
    xn, sn = _layer_update(x_ref[...], s_ref[...], tp_ref[...],
                           a0_ref[...] + a1_ref[...],
                           dinv, w1[...], b1[...], w2[...], b2[...], bg[...])
    stn = sn @ wgn[...]
    xo_ref[...] = xn
    so_ref[...] = sn
    to_ref[...] = stn * dinv


def _tc_layer(x, s, tp, a0, a1, dg, w1, b1, w2, b2, bg, wgn):
    row = lambda i: (i, 0)
    whole = lambda i: (0, 0)
    return pl.pallas_call(
        _lay_body,
        grid=(_GRID,),
        in_specs=[
            pl.BlockSpec((_BLK, 128), row),
            pl.BlockSpec((_BLK, 128), row),
            pl.BlockSpec((_BLK, 128), row),
            pl.BlockSpec((_BLK, W), row),
            pl.BlockSpec((_BLK, W), row),
            pl.BlockSpec((_BLK, 1), row),
            pl.BlockSpec((256, 128), whole),
            pl.BlockSpec((1, 128), whole),
            pl.BlockSpec((128, 128), whole),
            pl.BlockSpec((1, 128), whole),
            pl.BlockSpec((1, 128), whole),
            pl.BlockSpec((128, 128), whole),
        ],
        out_specs=[
            pl.BlockSpec((_BLK, 128), row),
            pl.BlockSpec((_BLK, 128), row),
            pl.BlockSpec((_BLK, 128), row),
        ],
        out_shape=[
            jax.ShapeDtypeStruct((NP, 128), F32),
            jax.ShapeDtypeStruct((NP, 128), F32),
            jax.ShapeDtypeStruct((NP, 128), F32),
        ],
    )(x, s, tp, a0, a1, dg, w1, b1, w2, b2, bg, wgn)


def _fin_body(x_ref, s_ref, tp_ref, a0_ref, a1_ref, dg, w1, b1, w2, b2, bg,
              whp, bhp, bat_ref, wpost, bpost, wro, bro,
              out_ref, pool_acc):
    i = pl.program_id(0)
    dinv = lax.rsqrt(dg[...] + 1.0)
    xn, sn = _layer_update(x_ref[...], s_ref[...], tp_ref[...],
                           a0_ref[...] + a1_ref[...],
                           dinv, w1[...], b1[...], w2[...], b2[...], bg[...])
    hp = jnp.concatenate([xn, sn], axis=1) @ whp[...] + bhp[...]
    onehot = (lax.broadcasted_iota(jnp.int32, (NGR, _BLK), 0)
              == bat_ref[0]).astype(F32)
    part = onehot @ hp

    @pl.when(i == 0)
    def _():
        pool_acc[...] = part

    @pl.when(i > 0)
    def _():
        pool_acc[...] = pool_acc[...] + part

    @pl.when(i == _GRID - 1)
    def _():
        p2 = jnp.maximum(pool_acc[...] @ wpost[...] + bpost[...], 0.0)
        logits = p2 @ wro[...] + bro[...]
        m = jnp.max(logits, axis=1, keepdims=True)
        z = logits - m
        lse = jnp.log(jnp.sum(jnp.exp(z), axis=1, keepdims=True))
        out_ref[...] = z - lse


def _tc_final(x, s, tp, a0, a1, dg, w1, b1, w2, b2, bg,
              whp, bhp, batr, wpost, bpost, wro, bro):
    row = lambda i: (i, 0)
    whole = lambda i: (0, 0)
    return pl.pallas_call(
        _fin_body,
        grid=(_GRID,),
        in_specs=[
            pl.BlockSpec((_BLK, 128), row),
            pl.BlockSpec((_BLK, 128), row),
            pl.BlockSpec((_BLK, 128), row),
            pl.BlockSpec((_BLK, W), row),
            pl.BlockSpec((_BLK, W), row),
            pl.BlockSpec((_BLK, 1), row),
            pl.BlockSpec((256, 128), whole),
            pl.BlockSpec((1, 128), whole),
            pl.BlockSpec((128, 128), whole),
            pl.BlockSpec((1, 128), whole),
            pl.BlockSpec((1, 128), whole),
            pl.BlockSpec((256, 128), whole),
            pl.BlockSpec((1, 128), whole),
            pl.BlockSpec((1, 1, _BLK), lambda i: (i, 0, 0)),
            pl.BlockSpec((128, 128), whole),
            pl.BlockSpec((1, 128), whole),
            pl.BlockSpec((128, NCLS), whole),
            pl.BlockSpec((1, NCLS), whole),
        ],
        out_specs=pl.BlockSpec((NGR, NCLS), whole),
        out_shape=jax.ShapeDtypeStruct((NGR, NCLS), F32),
        scratch_shapes=[pltpu.VMEM((NGR, 128), F32)],
    )(x, s, tp, a0, a1, dg, w1, b1, w2, b2, bg,
      whp, bhp, batr, wpost, bpost, wro, bro)


def kernel(x, s, W_pre, b_pre, W_es, b_es, W1s, b1s, W2s, b2s, Wgs, bgs,
           W_hp, b_hp, W_post, b_post, W_ro, b_ro, edge_index, batch):
    src = edge_index[0]
    dst = edge_index[1]
    npad = EP - E
    pad_src = jnp.arange(npad, dtype=jnp.int32) % 2048
    pad_dst = N + jnp.arange(npad, dtype=jnp.int32) % (NP - N)
    src_r = jnp.concatenate([src, pad_src]).reshape(2, NT, NCHUNK, CH)
    dst_r = jnp.concatenate([dst, pad_dst]).reshape(2, NT, NCHUNK, CH)
    zeros_t = jnp.zeros((RPT, 128), F32)

    xp = jnp.pad(x, ((0, NP - N), (0, 0)))
    sp = jnp.pad(s, ((0, NP - N), (0, 0)))
    batr = jnp.pad(batch, (0, NP - N),
                   constant_values=NGR).reshape(_GRID, 1, _BLK)

    degp = _sc_deg(dst_r).reshape(2, NP)
    dg = (degp[0] + degp[1])[:, None]

    r1 = lambda a: a.reshape(1, -1)
    xc, sc, t2 = _tc_pre(xp, sp, W_pre, r1(b_pre), W_es, r1(b_es),
                         Wgs[0], dg)
    for i in range(3):
        aggp = _sc_agg(xc, sc, t2, src_r, dst_r, zeros_t)
        if i < 2:
            xc, sc, t2 = _tc_layer(
                xc, sc, t2, aggp[0], aggp[1], dg,
                W1s[i], r1(b1s[i]), W2s[i], r1(b2s[i]), r1(bgs[i]),
                Wgs[i + 1])
        else:
            out = _tc_final(
                xc, sc, t2, aggp[0], aggp[1], dg,
                W1s[i], r1(b1s[i]), W2s[i], r1(b2s[i]), r1(bgs[i]),
                W_hp, r1(b_hp), batr, W_post, r1(b_post), W_ro, r1(b_ro))
    return out


# speedup vs baseline: 6.9970x; 1.0539x over previous
import jax
import jax.numpy as jnp
from jax import lax
from jax.experimental import pallas as pl
from jax.experimental.pallas import tpu as pltpu
from jax.experimental.pallas import tpu_sc as plsc

N = 10000
NP = 10240
E = 320000
NT = 16
CH = 64
NCHUNK = 160
EP = 2 * NT * NCHUNK * CH
RPT = NP // NT
W = 384
NGR = 128
NCLS = 10
F32 = jnp.float32

_MESH = dict(core_axis_name="c", subcore_axis_name="s", num_cores=2,
             num_subcores=16)


def _deg_body(dst_hbm, out_hbm, dst_v, ones_v, zbuf, acc1):
    c = lax.axis_index("c")
    sid = lax.axis_index("s")
    pltpu.sync_copy(dst_hbm.at[c, sid], dst_v)
    for i in range(CH // 16):
        ones_v[pl.ds(i * 16, 16)] = jnp.full((16,), 1.0, F32)
    for i in range(RPT // 16):
        zbuf[pl.ds(i * 16, 16)] = jnp.zeros((16,), F32)
    pltpu.sync_copy(zbuf, acc1.at[pl.ds(sid * RPT, RPT)])
    plsc.subcore_barrier()

    def chunk(j, carry):
        pltpu.sync_copy(ones_v, acc1.at[dst_v.at[j]], add=True)
        return carry

    lax.fori_loop(0, NCHUNK, chunk, 0)
    plsc.subcore_barrier()
    off = pl.multiple_of(c * NP + sid * RPT, 128)
    pltpu.sync_copy(acc1.at[pl.ds(sid * RPT, RPT)],
                    out_hbm.at[pl.ds(off, RPT)])


def _sc_deg(dst_r):
    k = pl.kernel(
        _deg_body,
        out_type=jax.ShapeDtypeStruct((2 * NP,), F32),
        mesh=plsc.VectorSubcoreMesh(**_MESH),
        scratch_types=[
            pltpu.VMEM((NCHUNK, CH), jnp.int32),
            pltpu.VMEM((CH,), F32),
            pltpu.VMEM((RPT,), F32),
            pltpu.VMEM_SHARED((NP,), F32),
        ],
    )
    return k(dst_r)


HC = NCHUNK // 2


def _agg_body(t0_hbm, t1_hbm, t2_hbm, src_hbm, dst_hbm, z_hbm, out_hbm,
              src_v, dst_v, rows_v, rows_b, sem, sem_b, ssem, ssem_b, acc):
    c = lax.axis_index("c")
    sid = lax.axis_index("s")
    for k, tab in enumerate((t0_hbm, t1_hbm, t2_hbm)):
        pltpu.sync_copy(z_hbm, acc.at[pl.ds(sid * RPT, RPT)])
        plsc.subcore_barrier()

        def wait_rows(buf, sem_, tab=tab):
            pltpu.make_async_copy(tab.at[src_v.at[0]], buf, sem_).wait()

        def wait_scat(buf, sem_):
            pltpu.make_async_copy(buf, acc.at[dst_v.at[0]], sem_).wait()

        for half in range(2):
            pltpu.sync_copy(src_hbm.at[c, sid, pl.ds(half * HC, HC)], src_v)
            pltpu.sync_copy(dst_hbm.at[c, sid, pl.ds(half * HC, HC)], dst_v)

            pltpu.async_copy(tab.at[src_v.at[0]], rows_v, sem)
            pltpu.async_copy(tab.at[src_v.at[1]], rows_b, sem_b)

            def pair(t, carry, tab=tab):
                j = t * 2
                wait_rows(rows_v, sem)
                pltpu.async_copy(rows_v, acc.at[dst_v.at[j]], ssem, add=True)
                wait_rows(rows_b, sem_b)
                pltpu.async_copy(rows_b, acc.at[dst_v.at[j + 1]], ssem_b,
                                 add=True)
                wait_scat(rows_v, ssem)
                wait_scat(rows_b, ssem_b)

                @pl.when(t < HC // 2 - 1)
                def _():
                    pltpu.async_copy(tab.at[src_v.at[j + 2]], rows_v, sem)
                    pltpu.async_copy(tab.at[src_v.at[j + 3]], rows_b, sem_b)

                return carry

            lax.fori_loop(0, HC // 2, pair, 0)
        plsc.subcore_barrier()
        pltpu.sync_copy(acc.at[pl.ds(sid * RPT, RPT)],
                        out_hbm.at[c, pl.ds(sid * RPT, RPT),
                                   pl.ds(k * 128, 128)])


def _sc_agg(t0, t1, t2, src_r, dst_r, zeros_t):
    k = pl.kernel(
        _agg_body,
        out_type=jax.ShapeDtypeStruct((2, NP, W), F32),
        mesh=plsc.VectorSubcoreMesh(**_MESH),
        scratch_types=[
            pltpu.VMEM((HC, CH), jnp.int32),
            pltpu.VMEM((HC, CH), jnp.int32),
            pltpu.VMEM((CH, 128), F32),
            pltpu.VMEM((CH, 128), F32),
            pltpu.SemaphoreType.DMA,
            pltpu.SemaphoreType.DMA,
            pltpu.SemaphoreType.DMA,
            pltpu.SemaphoreType.DMA,
            pltpu.VMEM_SHARED((NP, 128), F32),
        ],
    )
    return k(t0, t1, t2, src_r, dst_r, zeros_t)


_BLK = 1280
_GRID = NP // _BLK


def _pre_body(xp_ref, sp_ref, wpre, bpre, wes, bes, wg, dg,
              x1_ref, s1_ref, t2_ref):
    xv = xp_ref[...] @ wpre[...] + bpre[...]
    sv = sp_ref[...] @ wes[...] + bes[...]
    dinv = lax.rsqrt(dg[...] + 1.0)
    st = sv @ wg[...]
    x1_ref[...] = xv
    s1_ref[...] = sv
    t2_ref[...] = st * dinv


def _tc_pre(xp, sp, wpre, bpre, wes, bes, wg, dg):
    row = lambda i: (i, 0)
    whole = lambda i: (0, 0)
    return pl.pallas_call(
        _pre_body,
        grid=(_GRID,),
        in_specs=[
            pl.BlockSpec((_BLK, 128), row),
            pl.BlockSpec((_BLK, 16), row),
            pl.BlockSpec((128, 128), whole),
            pl.BlockSpec((1, 128), whole),
            pl.BlockSpec((16, 128), whole),
            pl.BlockSpec((1, 128), whole),
            pl.BlockSpec((128, 128), whole),
            pl.BlockSpec((_BLK, 1), row),
        ],
        out_specs=[
            pl.BlockSpec((_BLK, 128), row),
            pl.BlockSpec((_BLK, 128), row),
            pl.BlockSpec((_BLK, 128), row),
        ],
        out_shape=[
            jax.ShapeDtypeStruct((NP, 128), F32),
            jax.ShapeDtypeStruct((NP, 128), F32),
            jax.ShapeDtypeStruct((NP, 128), F32),
        ],
    )(xp, sp, wpre, bpre, wes, bes, wg, dg)


def _layer_update(x, s, t2prev, a, dinv, w1, b1, w2, b2, bg):
    hx = x + a[:, :128]
    hs = s + a[:, 128:256]
    h = jnp.concatenate([hx, hs], axis=1)
    m = jnp.maximum(h @ w1 + b1, 0.0) @ w2 + b2
    xn = jnp.maximum(m, 0.0)
    sn = jnp.tanh(dinv * (a[:, 256:] + t2prev) + bg)
    return xn, sn


def _lay_body(x_ref, s_ref, tp_ref, a0_ref, a1_ref, dg, w1, b1, w2, b2, bg,
              wgn, xo_ref, so_ref, to_ref):
    dinv = lax.rsqrt(dg[...] + 1.0)
    xn, sn = _layer_update(x_ref[...], s_ref[...], tp_ref[...],
                           a0_ref[...] + a1_ref[...],
                           dinv, w1[...], b1[...], w2[...], b2[...], bg[...])
    stn = sn @ wgn[...]
    xo_ref[...] = xn
    so_ref[...] = sn
    to_ref[...] = stn * dinv


def _tc_layer(x, s, tp, a0, a1, dg, w1, b1, w2, b2, bg, wgn):
    row = lambda i: (i, 0)
    whole = lambda i: (0, 0)
    return pl.pallas_call(
        _lay_body,
        grid=(_GRID,),
        in_specs=[
            pl.BlockSpec((_BLK, 128), row),
            pl.BlockSpec((_BLK, 128), row),
            pl.BlockSpec((_BLK, 128), row),
            pl.BlockSpec((_BLK, W), row),
            pl.BlockSpec((_BLK, W), row),
            pl.BlockSpec((_BLK, 1), row),
            pl.BlockSpec((256, 128), whole),
            pl.BlockSpec((1, 128), whole),
            pl.BlockSpec((128, 128), whole),
            pl.BlockSpec((1, 128), whole),
            pl.BlockSpec((1, 128), whole),
            pl.BlockSpec((128, 128), whole),
        ],
        out_specs=[
            pl.BlockSpec((_BLK, 128), row),
            pl.BlockSpec((_BLK, 128), row),
            pl.BlockSpec((_BLK, 128), row),
        ],
        out_shape=[
            jax.ShapeDtypeStruct((NP, 128), F32),
            jax.ShapeDtypeStruct((NP, 128), F32),
            jax.ShapeDtypeStruct((NP, 128), F32),
        ],
    )(x, s, tp, a0, a1, dg, w1, b1, w2, b2, bg, wgn)


def _fin_body(x_ref, s_ref, tp_ref, a0_ref, a1_ref, dg, w1, b1, w2, b2, bg,
              whp, bhp, bat_ref, wpost, bpost, wro, bro,
              out_ref, pool_acc):
    i = pl.program_id(0)
    dinv = lax.rsqrt(dg[...] + 1.0)
    xn, sn = _layer_update(x_ref[...], s_ref[...], tp_ref[...],
                           a0_ref[...] + a1_ref[...],
                           dinv, w1[...], b1[...], w2[...], b2[...], bg[...])
    hp = jnp.concatenate([xn, sn], axis=1) @ whp[...] + bhp[...]
    onehot = (lax.broadcasted_iota(jnp.int32, (NGR, _BLK), 0)
              == bat_ref[0]).astype(F32)
    part = onehot @ hp

    @pl.when(i == 0)
    def _():
        pool_acc[...] = part

    @pl.when(i > 0)
    def _():
        pool_acc[...] = pool_acc[...] + part

    @pl.when(i == _GRID - 1)
    def _():
        p2 = jnp.maximum(pool_acc[...] @ wpost[...] + bpost[...], 0.0)
        logits = p2 @ wro[...] + bro[...]
        m = jnp.max(logits, axis=1, keepdims=True)
        z = logits - m
        lse = jnp.log(jnp.sum(jnp.exp(z), axis=1, keepdims=True))
        out_ref[...] = z - lse


def _tc_final(x, s, tp, a0, a1, dg, w1, b1, w2, b2, bg,
              whp, bhp, batr, wpost, bpost, wro, bro):
    row = lambda i: (i, 0)
    whole = lambda i: (0, 0)
    return pl.pallas_call(
        _fin_body,
        grid=(_GRID,),
        in_specs=[
            pl.BlockSpec((_BLK, 128), row),
            pl.BlockSpec((_BLK, 128), row),
            pl.BlockSpec((_BLK, 128), row),
            pl.BlockSpec((_BLK, W), row),
            pl.BlockSpec((_BLK, W), row),
            pl.BlockSpec((_BLK, 1), row),
            pl.BlockSpec((256, 128), whole),
            pl.BlockSpec((1, 128), whole),
            pl.BlockSpec((128, 128), whole),
            pl.BlockSpec((1, 128), whole),
            pl.BlockSpec((1, 128), whole),
            pl.BlockSpec((256, 128), whole),
            pl.BlockSpec((1, 128), whole),
            pl.BlockSpec((1, 1, _BLK), lambda i: (i, 0, 0)),
            pl.BlockSpec((128, 128), whole),
            pl.BlockSpec((1, 128), whole),
            pl.BlockSpec((128, NCLS), whole),
            pl.BlockSpec((1, NCLS), whole),
        ],
        out_specs=pl.BlockSpec((NGR, NCLS), whole),
        out_shape=jax.ShapeDtypeStruct((NGR, NCLS), F32),
        scratch_shapes=[pltpu.VMEM((NGR, 128), F32)],
    )(x, s, tp, a0, a1, dg, w1, b1, w2, b2, bg,
      whp, bhp, batr, wpost, bpost, wro, bro)


def kernel(x, s, W_pre, b_pre, W_es, b_es, W1s, b1s, W2s, b2s, Wgs, bgs,
           W_hp, b_hp, W_post, b_post, W_ro, b_ro, edge_index, batch):
    src = edge_index[0]
    dst = edge_index[1]
    npad = EP - E
    pad_src = jnp.arange(npad, dtype=jnp.int32) % 2048
    pad_dst = N + jnp.arange(npad, dtype=jnp.int32) % (NP - N)
    src_r = jnp.concatenate([src, pad_src]).reshape(2, NT, NCHUNK, CH)
    dst_r = jnp.concatenate([dst, pad_dst]).reshape(2, NT, NCHUNK, CH)
    zeros_t = jnp.zeros((RPT, 128), F32)

    xp = jnp.pad(x, ((0, NP - N), (0, 0)))
    sp = jnp.pad(s, ((0, NP - N), (0, 0)))
    batr = jnp.pad(batch, (0, NP - N),
                   constant_values=NGR).reshape(_GRID, 1, _BLK)

    degp = _sc_deg(dst_r).reshape(2, NP)
    dg = (degp[0] + degp[1])[:, None]

    r1 = lambda a: a.reshape(1, -1)
    xc, sc, t2 = _tc_pre(xp, sp, W_pre, r1(b_pre), W_es, r1(b_es),
                         Wgs[0], dg)
    for i in range(3):
        aggp = _sc_agg(xc, sc, t2, src_r, dst_r, zeros_t)
        if i < 2:
            xc, sc, t2 = _tc_layer(
                xc, sc, t2, aggp[0], aggp[1], dg,
                W1s[i], r1(b1s[i]), W2s[i], r1(b2s[i]), r1(bgs[i]),
                Wgs[i + 1])
        else:
            out = _tc_final(
                xc, sc, t2, aggp[0], aggp[1], dg,
                W1s[i], r1(b1s[i]), W2s[i], r1(b2s[i]), r1(bgs[i]),
                W_hp, r1(b_hp), batr, W_post, r1(b_post), W_ro, r1(b_ro))
    return out


# speedup vs baseline: 7.8536x; 1.1224x over previous
import jax
import jax.numpy as jnp
from jax import lax
from jax.experimental import pallas as pl
from jax.experimental.pallas import tpu as pltpu
from jax.experimental.pallas import tpu_sc as plsc

N = 10000
NP = 10240
E = 320000
NT = 16
CH = 128
NCHUNK = 80
NSEG = 2
EP = 2 * NT * NCHUNK * CH
RPT = NP // NT
W = 384
NGR = 128
NCLS = 10
F32 = jnp.float32

_MESH = dict(core_axis_name="c", subcore_axis_name="s", num_cores=2,
             num_subcores=16)


def _deg_body(dst_hbm, out_hbm, dst_v, ones_v, zbuf, acc1):
    c = lax.axis_index("c")
    sid = lax.axis_index("s")
    pltpu.sync_copy(dst_hbm.at[c, sid], dst_v)
    for i in range(CH // 16):
        ones_v[pl.ds(i * 16, 16)] = jnp.full((16,), 1.0, F32)
    for i in range(RPT // 16):
        zbuf[pl.ds(i * 16, 16)] = jnp.zeros((16,), F32)
    pltpu.sync_copy(zbuf, acc1.at[pl.ds(sid * RPT, RPT)])
    plsc.subcore_barrier()

    def chunk(j, carry):
        pltpu.sync_copy(ones_v, acc1.at[dst_v.at[j]], add=True)
        return carry

    lax.fori_loop(0, NCHUNK, chunk, 0)
    plsc.subcore_barrier()
    off = pl.multiple_of(c * NP + sid * RPT, 128)
    pltpu.sync_copy(acc1.at[pl.ds(sid * RPT, RPT)],
                    out_hbm.at[pl.ds(off, RPT)])


def _sc_deg(dst_r):
    k = pl.kernel(
        _deg_body,
        out_type=jax.ShapeDtypeStruct((2 * NP,), F32),
        mesh=plsc.VectorSubcoreMesh(**_MESH),
        scratch_types=[
            pltpu.VMEM((NCHUNK, CH), jnp.int32),
            pltpu.VMEM((CH,), F32),
            pltpu.VMEM((RPT,), F32),
            pltpu.VMEM_SHARED((NP,), F32),
        ],
    )
    return k(dst_r)


HC = NCHUNK // NSEG


def _agg_body(t0_hbm, t1_hbm, t2_hbm, src_hbm, dst_hbm, z_hbm, out_hbm,
              src_v, dst_v, rows_v, rows_b, sem, sem_b, ssem, ssem_b, acc):
    c = lax.axis_index("c")
    sid = lax.axis_index("s")
    for k, tab in enumerate((t0_hbm, t1_hbm, t2_hbm)):
        pltpu.sync_copy(z_hbm, acc.at[pl.ds(sid * RPT, RPT)])
        plsc.subcore_barrier()

        def wait_rows(buf, sem_, tab=tab):
            pltpu.make_async_copy(tab.at[src_v.at[0]], buf, sem_).wait()

        def wait_scat(buf, sem_):
            pltpu.make_async_copy(buf, acc.at[dst_v.at[0]], sem_).wait()

        for half in range(NSEG):
            pltpu.sync_copy(src_hbm.at[c, sid, pl.ds(half * HC, HC)], src_v)
            pltpu.sync_copy(dst_hbm.at[c, sid, pl.ds(half * HC, HC)], dst_v)

            pltpu.async_copy(tab.at[src_v.at[0]], rows_v, sem)
            pltpu.async_copy(tab.at[src_v.at[1]], rows_b, sem_b)

            def pair(t, carry, tab=tab):
                j = t * 2
                wait_rows(rows_v, sem)
                pltpu.async_copy(rows_v, acc.at[dst_v.at[j]], ssem, add=True)
                wait_rows(rows_b, sem_b)
                pltpu.async_copy(rows_b, acc.at[dst_v.at[j + 1]], ssem_b,
                                 add=True)
                wait_scat(rows_v, ssem)
                wait_scat(rows_b, ssem_b)

                @pl.when(t < HC // 2 - 1)
                def _():
                    pltpu.async_copy(tab.at[src_v.at[j + 2]], rows_v, sem)
                    pltpu.async_copy(tab.at[src_v.at[j + 3]], rows_b, sem_b)

                return carry

            lax.fori_loop(0, HC // 2, pair, 0)
        plsc.subcore_barrier()
        pltpu.sync_copy(acc.at[pl.ds(sid * RPT, RPT)],
                        out_hbm.at[c, pl.ds(sid * RPT, RPT),
                                   pl.ds(k * 128, 128)])


def _sc_agg(t0, t1, t2, src_r, dst_r, zeros_t):
    k = pl.kernel(
        _agg_body,
        out_type=jax.ShapeDtypeStruct((2, NP, W), F32),
        mesh=plsc.VectorSubcoreMesh(**_MESH),
        scratch_types=[
            pltpu.VMEM((HC, CH), jnp.int32),
            pltpu.VMEM((HC, CH), jnp.int32),
            pltpu.VMEM((CH, 128), F32),
            pltpu.VMEM((CH, 128), F32),
            pltpu.SemaphoreType.DMA,
            pltpu.SemaphoreType.DMA,
            pltpu.SemaphoreType.DMA,
            pltpu.SemaphoreType.DMA,
            pltpu.VMEM_SHARED((NP, 128), F32),
        ],
    )
    return k(t0, t1, t2, src_r, dst_r, zeros_t)


_BLK = 1280
_GRID = NP // _BLK


def _pre_body(xp_ref, sp_ref, wpre, bpre, wes, bes, wg, dg,
              x1_ref, s1_ref, t2_ref):
    xv = xp_ref[...] @ wpre[...] + bpre[...]
    sv = sp_ref[...] @ wes[...] + bes[...]
    dinv = lax.rsqrt(dg[...] + 1.0)
    st = sv @ wg[...]
    x1_ref[...] = xv
    s1_ref[...] = sv
    t2_ref[...] = st * dinv


def _tc_pre(xp, sp, wpre, bpre, wes, bes, wg, dg):
    row = lambda i: (i, 0)
    whole = lambda i: (0, 0)
    return pl.pallas_call(
        _pre_body,
        grid=(_GRID,),
        in_specs=[
            pl.BlockSpec((_BLK, 128), row),
            pl.BlockSpec((_BLK, 16), row),
            pl.BlockSpec((128, 128), whole),
            pl.BlockSpec((1, 128), whole),
            pl.BlockSpec((16, 128), whole),
            pl.BlockSpec((1, 128), whole),
            pl.BlockSpec((128, 128), whole),
            pl.BlockSpec((_BLK, 1), row),
        ],
        out_specs=[
            pl.BlockSpec((_BLK, 128), row),
            pl.BlockSpec((_BLK, 128), row),
            pl.BlockSpec((_BLK, 128), row),
        ],
        out_shape=[
            jax.ShapeDtypeStruct((NP, 128), F32),
            jax.ShapeDtypeStruct((NP, 128), F32),
            jax.ShapeDtypeStruct((NP, 128), F32),
        ],
    )(xp, sp, wpre, bpre, wes, bes, wg, dg)


def _layer_update(x, s, t2prev, a, dinv, w1, b1, w2, b2, bg):
    hx = x + a[:, :128]
    hs = s + a[:, 128:256]
    h = jnp.concatenate([hx, hs], axis=1)
    m = jnp.maximum(h @ w1 + b1, 0.0) @ w2 + b2
    xn = jnp.maximum(m, 0.0)
    sn = jnp.tanh(dinv * (a[:, 256:] + t2prev) + bg)
    return xn, sn


def _lay_body(x_ref, s_ref, tp_ref, a0_ref, a1_ref, dg, w1, b1, w2, b2, bg,
              wgn, xo_ref, so_ref, to_ref):
    dinv = lax.rsqrt(dg[...] + 1.0)
    xn, sn = _layer_update(x_ref[...], s_ref[...], tp_ref[...],
                           a0_ref[...] + a1_ref[...],
                           dinv, w1[...], b1[...], w2[...], b2[...], bg[...])
    stn = sn @ wgn[...]
    xo_ref[...] = xn
    so_ref[...] = sn
    to_ref[...] = stn * dinv


def _tc_layer(x, s, tp, a0, a1, dg, w1, b1, w2, b2, bg, wgn):
    row = lambda i: (i, 0)
    whole = lambda i: (0, 0)
    return pl.pallas_call(
        _lay_body,
        grid=(_GRID,),
        in_specs=[
            pl.BlockSpec((_BLK, 128), row),
            pl.BlockSpec((_BLK, 128), row),
            pl.BlockSpec((_BLK, 128), row),
            pl.BlockSpec((_BLK, W), row),
            pl.BlockSpec((_BLK, W), row),
            pl.BlockSpec((_BLK, 1), row),
            pl.BlockSpec((256, 128), whole),
            pl.BlockSpec((1, 128), whole),
            pl.BlockSpec((128, 128), whole),
            pl.BlockSpec((1, 128), whole),
            pl.BlockSpec((1, 128), whole),
            pl.BlockSpec((128, 128), whole),
        ],
        out_specs=[
            pl.BlockSpec((_BLK, 128), row),
            pl.BlockSpec((_BLK, 128), row),
            pl.BlockSpec((_BLK, 128), row),
        ],
        out_shape=[
            jax.ShapeDtypeStruct((NP, 128), F32),
            jax.ShapeDtypeStruct((NP, 128), F32),
            jax.ShapeDtypeStruct((NP, 128), F32),
        ],
    )(x, s, tp, a0, a1, dg, w1, b1, w2, b2, bg, wgn)


def _fin_body(x_ref, s_ref, tp_ref, a0_ref, a1_ref, dg, w1, b1, w2, b2, bg,
              whp, bhp, bat_ref, wpost, bpost, wro, bro,
              out_ref, pool_acc):
    i = pl.program_id(0)
    dinv = lax.rsqrt(dg[...] + 1.0)
    xn, sn = _layer_update(x_ref[...], s_ref[...], tp_ref[...],
                           a0_ref[...] + a1_ref[...],
                           dinv, w1[...], b1[...], w2[...], b2[...], bg[...])
    hp = jnp.concatenate([xn, sn], axis=1) @ whp[...] + bhp[...]
    onehot = (lax.broadcasted_iota(jnp.int32, (NGR, _BLK), 0)
              == bat_ref[0]).astype(F32)
    part = onehot @ hp

    @pl.when(i == 0)
    def _():
        pool_acc[...] = part

    @pl.when(i > 0)
    def _():
        pool_acc[...] = pool_acc[...] + part

    @pl.when(i == _GRID - 1)
    def _():
        p2 = jnp.maximum(pool_acc[...] @ wpost[...] + bpost[...], 0.0)
        logits = p2 @ wro[...] + bro[...]
        m = jnp.max(logits, axis=1, keepdims=True)
        z = logits - m
        lse = jnp.log(jnp.sum(jnp.exp(z), axis=1, keepdims=True))
        out_ref[...] = z - lse


def _tc_final(x, s, tp, a0, a1, dg, w1, b1, w2, b2, bg,
              whp, bhp, batr, wpost, bpost, wro, bro):
    row = lambda i: (i, 0)
    whole = lambda i: (0, 0)
    return pl.pallas_call(
        _fin_body,
        grid=(_GRID,),
        in_specs=[
            pl.BlockSpec((_BLK, 128), row),
            pl.BlockSpec((_BLK, 128), row),
            pl.BlockSpec((_BLK, 128), row),
            pl.BlockSpec((_BLK, W), row),
            pl.BlockSpec((_BLK, W), row),
            pl.BlockSpec((_BLK, 1), row),
            pl.BlockSpec((256, 128), whole),
            pl.BlockSpec((1, 128), whole),
            pl.BlockSpec((128, 128), whole),
            pl.BlockSpec((1, 128), whole),
            pl.BlockSpec((1, 128), whole),
            pl.BlockSpec((256, 128), whole),
            pl.BlockSpec((1, 128), whole),
            pl.BlockSpec((1, 1, _BLK), lambda i: (i, 0, 0)),
            pl.BlockSpec((128, 128), whole),
            pl.BlockSpec((1, 128), whole),
            pl.BlockSpec((128, NCLS), whole),
            pl.BlockSpec((1, NCLS), whole),
        ],
        out_specs=pl.BlockSpec((NGR, NCLS), whole),
        out_shape=jax.ShapeDtypeStruct((NGR, NCLS), F32),
        scratch_shapes=[pltpu.VMEM((NGR, 128), F32)],
    )(x, s, tp, a0, a1, dg, w1, b1, w2, b2, bg,
      whp, bhp, batr, wpost, bpost, wro, bro)


def kernel(x, s, W_pre, b_pre, W_es, b_es, W1s, b1s, W2s, b2s, Wgs, bgs,
           W_hp, b_hp, W_post, b_post, W_ro, b_ro, edge_index, batch):
    src = edge_index[0]
    dst = edge_index[1]
    npad = EP - E
    pad_src = jnp.arange(npad, dtype=jnp.int32) % 2048
    pad_dst = N + jnp.arange(npad, dtype=jnp.int32) % (NP - N)
    src_r = jnp.concatenate([src, pad_src]).reshape(2, NT, NCHUNK, CH)
    dst_r = jnp.concatenate([dst, pad_dst]).reshape(2, NT, NCHUNK, CH)
    zeros_t = jnp.zeros((RPT, 128), F32)

    xp = jnp.pad(x, ((0, NP - N), (0, 0)))
    sp = jnp.pad(s, ((0, NP - N), (0, 0)))
    batr = jnp.pad(batch, (0, NP - N),
                   constant_values=NGR).reshape(_GRID, 1, _BLK)

    degp = _sc_deg(dst_r).reshape(2, NP)
    dg = (degp[0] + degp[1])[:, None]

    r1 = lambda a: a.reshape(1, -1)
    xc, sc, t2 = _tc_pre(xp, sp, W_pre, r1(b_pre), W_es, r1(b_es),
                         Wgs[0], dg)
    for i in range(3):
        aggp = _sc_agg(xc, sc, t2, src_r, dst_r, zeros_t)
        if i < 2:
            xc, sc, t2 = _tc_layer(
                xc, sc, t2, aggp[0], aggp[1], dg,
                W1s[i], r1(b1s[i]), W2s[i], r1(b2s[i]), r1(bgs[i]),
                Wgs[i + 1])
        else:
            out = _tc_final(
                xc, sc, t2, aggp[0], aggp[1], dg,
                W1s[i], r1(b1s[i]), W2s[i], r1(b2s[i]), r1(bgs[i]),
                W_hp, r1(b_hp), batr, W_post, r1(b_post), W_ro, r1(b_ro))
    return out


# speedup vs baseline: 11.3847x; 1.4496x over previous
import jax
import jax.numpy as jnp
from jax import lax
from jax.experimental import pallas as pl
from jax.experimental.pallas import tpu as pltpu
from jax.experimental.pallas import tpu_sc as plsc

N = 10000
NP = 10240
E = 320000
NT = 16
CH = 128
NCHUNK = 80
NSEG = 2
EP = 2 * NT * NCHUNK * CH
RPT = NP // NT
W = 256
NGR = 128
NCLS = 10
F32 = jnp.float32

_MESH = dict(core_axis_name="c", subcore_axis_name="s", num_cores=2,
             num_subcores=16)


def _deg_body(dst_hbm, out_hbm, dst_v, ones_v, zbuf, acc1):
    c = lax.axis_index("c")
    sid = lax.axis_index("s")
    pltpu.sync_copy(dst_hbm.at[c, sid], dst_v)
    for i in range(CH // 16):
        ones_v[pl.ds(i * 16, 16)] = jnp.full((16,), 1.0, F32)
    for i in range(RPT // 16):
        zbuf[pl.ds(i * 16, 16)] = jnp.zeros((16,), F32)
    pltpu.sync_copy(zbuf, acc1.at[pl.ds(sid * RPT, RPT)])
    plsc.subcore_barrier()

    def chunk(j, carry):
        pltpu.sync_copy(ones_v, acc1.at[dst_v.at[j]], add=True)
        return carry

    lax.fori_loop(0, NCHUNK, chunk, 0)
    plsc.subcore_barrier()
    off = pl.multiple_of(c * NP + sid * RPT, 128)
    pltpu.sync_copy(acc1.at[pl.ds(sid * RPT, RPT)],
                    out_hbm.at[pl.ds(off, RPT)])


def _sc_deg(dst_r):
    k = pl.kernel(
        _deg_body,
        out_type=jax.ShapeDtypeStruct((2 * NP,), F32),
        mesh=plsc.VectorSubcoreMesh(**_MESH),
        scratch_types=[
            pltpu.VMEM((NCHUNK, CH), jnp.int32),
            pltpu.VMEM((CH,), F32),
            pltpu.VMEM((RPT,), F32),
            pltpu.VMEM_SHARED((NP,), F32),
        ],
    )
    return k(dst_r)


HC = NCHUNK // NSEG


def _agg_body(t0_hbm, t1_hbm, src_hbm, dst_hbm, z_hbm, out_hbm,
              src_v, dst_v, rows_v, rows_b, sem, sem_b, ssem, ssem_b, acc):
    c = lax.axis_index("c")
    sid = lax.axis_index("s")
    for k, tab in enumerate((t0_hbm, t1_hbm)):
        pltpu.sync_copy(z_hbm, acc.at[pl.ds(sid * RPT, RPT)])
        plsc.subcore_barrier()

        def wait_rows(buf, sem_, tab=tab):
            pltpu.make_async_copy(tab.at[src_v.at[0]], buf, sem_).wait()

        def wait_scat(buf, sem_):
            pltpu.make_async_copy(buf, acc.at[dst_v.at[0]], sem_).wait()

        for half in range(NSEG):
            pltpu.sync_copy(src_hbm.at[c, sid, pl.ds(half * HC, HC)], src_v)
            pltpu.sync_copy(dst_hbm.at[c, sid, pl.ds(half * HC, HC)], dst_v)

            pltpu.async_copy(tab.at[src_v.at[0]], rows_v, sem)
            pltpu.async_copy(tab.at[src_v.at[1]], rows_b, sem_b)

            def pair(t, carry, tab=tab):
                j = t * 2
                wait_rows(rows_v, sem)
                pltpu.async_copy(rows_v, acc.at[dst_v.at[j]], ssem, add=True)
                wait_rows(rows_b, sem_b)
                pltpu.async_copy(rows_b, acc.at[dst_v.at[j + 1]], ssem_b,
                                 add=True)
                wait_scat(rows_v, ssem)
                wait_scat(rows_b, ssem_b)

                @pl.when(t < HC // 2 - 1)
                def _():
                    pltpu.async_copy(tab.at[src_v.at[j + 2]], rows_v, sem)
                    pltpu.async_copy(tab.at[src_v.at[j + 3]], rows_b, sem_b)

                return carry

            lax.fori_loop(0, HC // 2, pair, 0)
        plsc.subcore_barrier()
        pltpu.sync_copy(acc.at[pl.ds(sid * RPT, RPT)],
                        out_hbm.at[c, pl.ds(sid * RPT, RPT),
                                   pl.ds(k * 128, 128)])


def _sc_agg(t0, t1, src_r, dst_r, zeros_t):
    k = pl.kernel(
        _agg_body,
        out_type=jax.ShapeDtypeStruct((2, NP, W), F32),
        mesh=plsc.VectorSubcoreMesh(**_MESH),
        scratch_types=[
            pltpu.VMEM((HC, CH), jnp.int32),
            pltpu.VMEM((HC, CH), jnp.int32),
            pltpu.VMEM((CH, 128), F32),
            pltpu.VMEM((CH, 128), F32),
            pltpu.SemaphoreType.DMA,
            pltpu.SemaphoreType.DMA,
            pltpu.SemaphoreType.DMA,
            pltpu.SemaphoreType.DMA,
            pltpu.VMEM_SHARED((NP, 128), F32),
        ],
    )
    return k(t0, t1, src_r, dst_r, zeros_t)


_BLK = 1280
_GRID = NP // _BLK


def _pre_body(xp_ref, sp_ref, wpre, bpre, wes, bes, w1a, w1b, dg,
              z_ref, u_ref):
    xv = xp_ref[...] @ wpre[...] + bpre[...]
    sv = sp_ref[...] @ wes[...] + bes[...]
    dinv = lax.rsqrt(dg[...] + 1.0)
    z_ref[...] = xv @ w1a[...] + sv @ w1b[...]
    u_ref[...] = sv * dinv


def _tc_pre(xp, sp, wpre, bpre, wes, bes, w1a, w1b, dg):
    row = lambda i: (i, 0)
    whole = lambda i: (0, 0)
    return pl.pallas_call(
        _pre_body,
        grid=(_GRID,),
        in_specs=[
            pl.BlockSpec((_BLK, 128), row),
            pl.BlockSpec((_BLK, 16), row),
            pl.BlockSpec((128, 128), whole),
            pl.BlockSpec((1, 128), whole),
            pl.BlockSpec((16, 128), whole),
            pl.BlockSpec((1, 128), whole),
            pl.BlockSpec((128, 128), whole),
            pl.BlockSpec((128, 128), whole),
            pl.BlockSpec((_BLK, 1), row),
        ],
        out_specs=[
            pl.BlockSpec((_BLK, 128), row),
            pl.BlockSpec((_BLK, 128), row),
        ],
        out_shape=[
            jax.ShapeDtypeStruct((NP, 128), F32),
            jax.ShapeDtypeStruct((NP, 128), F32),
        ],
    )(xp, sp, wpre, bpre, wes, bes, w1a, w1b, dg)


def _layer_update(z, u, a, dinv, b1, w2, b2, bg, wg):
    m = jnp.maximum(z + a[:, :128] + b1, 0.0) @ w2 + b2
    xn = jnp.maximum(m, 0.0)
    sn = jnp.tanh(dinv * ((a[:, 128:] + u) @ wg) + bg)
    return xn, sn


def _lay_body(z_ref, u_ref, a0_ref, a1_ref, dg, b1, w2, b2, bg, wg, w1n,
              zo_ref, uo_ref):
    dinv = lax.rsqrt(dg[...] + 1.0)
    xn, sn = _layer_update(z_ref[...], u_ref[...], a0_ref[...] + a1_ref[...],
                           dinv, b1[...], w2[...], b2[...], bg[...], wg[...])
    zo_ref[...] = jnp.concatenate([xn, sn], axis=1) @ w1n[...]
    uo_ref[...] = sn * dinv


def _tc_layer(z, u, a0, a1, dg, b1, w2, b2, bg, wg, w1n):
    row = lambda i: (i, 0)
    whole = lambda i: (0, 0)
    return pl.pallas_call(
        _lay_body,
        grid=(_GRID,),
        in_specs=[
            pl.BlockSpec((_BLK, 128), row),
            pl.BlockSpec((_BLK, 128), row),
            pl.BlockSpec((_BLK, W), row),
            pl.BlockSpec((_BLK, W), row),
            pl.BlockSpec((_BLK, 1), row),
            pl.BlockSpec((1, 128), whole),
            pl.BlockSpec((128, 128), whole),
            pl.BlockSpec((1, 128), whole),
            pl.BlockSpec((1, 128), whole),
            pl.BlockSpec((128, 128), whole),
            pl.BlockSpec((256, 128), whole),
        ],
        out_specs=[
            pl.BlockSpec((_BLK, 128), row),
            pl.BlockSpec((_BLK, 128), row),
        ],
        out_shape=[
            jax.ShapeDtypeStruct((NP, 128), F32),
            jax.ShapeDtypeStruct((NP, 128), F32),
        ],
    )(z, u, a0, a1, dg, b1, w2, b2, bg, wg, w1n)


def _fin_body(z_ref, u_ref, a0_ref, a1_ref, dg, b1, w2, b2, bg, wg,
              whp, bhp, bat_ref, wpost, bpost, wro, bro,
              out_ref, pool_acc):
    i = pl.program_id(0)
    dinv = lax.rsqrt(dg[...] + 1.0)
    xn, sn = _layer_update(z_ref[...], u_ref[...], a0_ref[...] + a1_ref[...],
                           dinv, b1[...], w2[...], b2[...], bg[...], wg[...])
    hp = jnp.concatenate([xn, sn], axis=1) @ whp[...] + bhp[...]
    onehot = (lax.broadcasted_iota(jnp.int32, (NGR, _BLK), 0)
              == bat_ref[0]).astype(F32)
    part = onehot @ hp

    @pl.when(i == 0)
    def _():
        pool_acc[...] = part

    @pl.when(i > 0)
    def _():
        pool_acc[...] = pool_acc[...] + part

    @pl.when(i == _GRID - 1)
    def _():
        p2 = jnp.maximum(pool_acc[...] @ wpost[...] + bpost[...], 0.0)
        logits = p2 @ wro[...] + bro[...]
        m = jnp.max(logits, axis=1, keepdims=True)
        z = logits - m
        lse = jnp.log(jnp.sum(jnp.exp(z), axis=1, keepdims=True))
        out_ref[...] = z - lse


def _tc_final(z, u, a0, a1, dg, b1, w2, b2, bg, wg,
              whp, bhp, batr, wpost, bpost, wro, bro):
    row = lambda i: (i, 0)
    whole = lambda i: (0, 0)
    return pl.pallas_call(
        _fin_body,
        grid=(_GRID,),
        in_specs=[
            pl.BlockSpec((_BLK, 128), row),
            pl.BlockSpec((_BLK, 128), row),
            pl.BlockSpec((_BLK, W), row),
            pl.BlockSpec((_BLK, W), row),
            pl.BlockSpec((_BLK, 1), row),
            pl.BlockSpec((1, 128), whole),
            pl.BlockSpec((128, 128), whole),
            pl.BlockSpec((1, 128), whole),
            pl.BlockSpec((1, 128), whole),
            pl.BlockSpec((128, 128), whole),
            pl.BlockSpec((256, 128), whole),
            pl.BlockSpec((1, 128), whole),
            pl.BlockSpec((1, 1, _BLK), lambda i: (i, 0, 0)),
            pl.BlockSpec((128, 128), whole),
            pl.BlockSpec((1, 128), whole),
            pl.BlockSpec((128, NCLS), whole),
            pl.BlockSpec((1, NCLS), whole),
        ],
        out_specs=pl.BlockSpec((NGR, NCLS), whole),
        out_shape=jax.ShapeDtypeStruct((NGR, NCLS), F32),
        scratch_shapes=[pltpu.VMEM((NGR, 128), F32)],
    )(z, u, a0, a1, dg, b1, w2, b2, bg, wg,
      whp, bhp, batr, wpost, bpost, wro, bro)


def kernel(x, s, W_pre, b_pre, W_es, b_es, W1s, b1s, W2s, b2s, Wgs, bgs,
           W_hp, b_hp, W_post, b_post, W_ro, b_ro, edge_index, batch):
    src = edge_index[0]
    dst = edge_index[1]
    npad = EP - E
    pad_src = jnp.arange(npad, dtype=jnp.int32) % 2048
    pad_dst = N + jnp.arange(npad, dtype=jnp.int32) % (NP - N)
    src_r = jnp.concatenate([src, pad_src]).reshape(2, NT, NCHUNK, CH)
    dst_r = jnp.concatenate([dst, pad_dst]).reshape(2, NT, NCHUNK, CH)
    zeros_t = jnp.zeros((RPT, 128), F32)

    xp = jnp.pad(x, ((0, NP - N), (0, 0)))
    sp = jnp.pad(s, ((0, NP - N), (0, 0)))
    batr = jnp.pad(batch, (0, NP - N),
                   constant_values=NGR).reshape(_GRID, 1, _BLK)

    degp = _sc_deg(dst_r).reshape(2, NP)
    dg = (degp[0] + degp[1])[:, None]

    r1 = lambda a: a.reshape(1, -1)
    zc, uc = _tc_pre(xp, sp, W_pre, r1(b_pre), W_es, r1(b_es),
                     W1s[0, :128], W1s[0, 128:], dg)
    for i in range(3):
        aggp = _sc_agg(zc, uc, src_r, dst_r, zeros_t)
        if i < 2:
            zc, uc = _tc_layer(
                zc, uc, aggp[0], aggp[1], dg,
                r1(b1s[i]), W2s[i], r1(b2s[i]), r1(bgs[i]), Wgs[i],
                W1s[i + 1])
        else:
            out = _tc_final(
                zc, uc, aggp[0], aggp[1], dg,
                r1(b1s[i]), W2s[i], r1(b2s[i]), r1(bgs[i]), Wgs[i],
                W_hp, r1(b_hp), batr, W_post, r1(b_post), W_ro, r1(b_ro))
    return out


# speedup vs baseline: 11.5540x; 1.0149x over previous
import jax
import jax.numpy as jnp
from jax import lax
from jax.experimental import pallas as pl
from jax.experimental.pallas import tpu as pltpu
from jax.experimental.pallas import tpu_sc as plsc

N = 10000
NP = 10240
E = 320000
NT = 16
CH = 128
NCHUNK = 80
NSEG = 2
EP = 2 * NT * NCHUNK * CH
RPT = NP // NT
W = 256
NGR = 128
NCLS = 10
F32 = jnp.float32

_MESH = dict(core_axis_name="c", subcore_axis_name="s", num_cores=2,
             num_subcores=16)


def _deg_body(dst_hbm, out_hbm, dst_v, ones_v, zbuf, acc1):
    c = lax.axis_index("c")
    sid = lax.axis_index("s")
    pltpu.sync_copy(dst_hbm.at[c, sid], dst_v)
    for i in range(CH // 16):
        ones_v[pl.ds(i * 16, 16)] = jnp.full((16,), 1.0, F32)
    for i in range(RPT // 16):
        zbuf[pl.ds(i * 16, 16)] = jnp.zeros((16,), F32)
    pltpu.sync_copy(zbuf, acc1.at[pl.ds(sid * RPT, RPT)])
    plsc.subcore_barrier()

    def chunk(j, carry):
        pltpu.sync_copy(ones_v, acc1.at[dst_v.at[j]], add=True)
        return carry

    lax.fori_loop(0, NCHUNK, chunk, 0)
    plsc.subcore_barrier()
    off = pl.multiple_of(c * NP + sid * RPT, 128)
    pltpu.sync_copy(acc1.at[pl.ds(sid * RPT, RPT)],
                    out_hbm.at[pl.ds(off, RPT)])


def _sc_deg(dst_r):
    k = pl.kernel(
        _deg_body,
        out_type=jax.ShapeDtypeStruct((2 * NP,), F32),
        mesh=plsc.VectorSubcoreMesh(**_MESH),
        scratch_types=[
            pltpu.VMEM((NCHUNK, CH), jnp.int32),
            pltpu.VMEM((CH,), F32),
            pltpu.VMEM((RPT,), F32),
            pltpu.VMEM_SHARED((NP,), F32),
        ],
    )
    return k(dst_r)


HC = NCHUNK // NSEG


def _agg_body(t0_hbm, t1_hbm, src_hbm, dst_hbm, z_hbm, out_hbm,
              src_v, dst_v, rows_v, rows_b, sem, sem_b, ssem, ssem_b, acc):
    c = lax.axis_index("c")
    sid = lax.axis_index("s")
    for k, tab in enumerate((t0_hbm, t1_hbm)):
        pltpu.sync_copy(z_hbm, acc.at[pl.ds(sid * RPT, RPT)])
        plsc.subcore_barrier()

        def wait_rows(buf, sem_, tab=tab):
            pltpu.make_async_copy(tab.at[src_v.at[0]], buf, sem_).wait()

        def wait_scat(buf, sem_):
            pltpu.make_async_copy(buf, acc.at[dst_v.at[0]], sem_).wait()

        for half in range(NSEG):
            pltpu.sync_copy(src_hbm.at[c, sid, pl.ds(half * HC, HC)], src_v)
            pltpu.sync_copy(dst_hbm.at[c, sid, pl.ds(half * HC, HC)], dst_v)

            pltpu.async_copy(tab.at[src_v.at[0]], rows_v, sem)
            pltpu.async_copy(tab.at[src_v.at[1]], rows_b, sem_b)

            def pair(t, carry, tab=tab):
                j = t * 2
                wait_rows(rows_v, sem)
                pltpu.async_copy(rows_v, acc.at[dst_v.at[j]], ssem, add=True)
                wait_rows(rows_b, sem_b)
                pltpu.async_copy(rows_b, acc.at[dst_v.at[j + 1]], ssem_b,
                                 add=True)
                wait_scat(rows_v, ssem)

                @pl.when(t < HC // 2 - 1)
                def _():
                    pltpu.async_copy(tab.at[src_v.at[j + 2]], rows_v, sem)

                wait_scat(rows_b, ssem_b)

                @pl.when(t < HC // 2 - 1)
                def _():
                    pltpu.async_copy(tab.at[src_v.at[j + 3]], rows_b, sem_b)

                return carry

            lax.fori_loop(0, HC // 2, pair, 0)
        plsc.subcore_barrier()
        pltpu.sync_copy(acc.at[pl.ds(sid * RPT, RPT)],
                        out_hbm.at[c, pl.ds(sid * RPT, RPT),
                                   pl.ds(k * 128, 128)])


def _sc_agg(t0, t1, src_r, dst_r, zeros_t):
    k = pl.kernel(
        _agg_body,
        out_type=jax.ShapeDtypeStruct((2, NP, W), F32),
        mesh=plsc.VectorSubcoreMesh(**_MESH),
        scratch_types=[
            pltpu.VMEM((HC, CH), jnp.int32),
            pltpu.VMEM((HC, CH), jnp.int32),
            pltpu.VMEM((CH, 128), F32),
            pltpu.VMEM((CH, 128), F32),
            pltpu.SemaphoreType.DMA,
            pltpu.SemaphoreType.DMA,
            pltpu.SemaphoreType.DMA,
            pltpu.SemaphoreType.DMA,
            pltpu.VMEM_SHARED((NP, 128), F32),
        ],
    )
    return k(t0, t1, src_r, dst_r, zeros_t)


_BLK = 1280
_GRID = NP // _BLK


def _pre_body(xp_ref, sp_ref, wpre, bpre, wes, bes, w1a, w1b, dg,
              z_ref, u_ref):
    xv = xp_ref[...] @ wpre[...] + bpre[...]
    sv = sp_ref[...] @ wes[...] + bes[...]
    dinv = lax.rsqrt(dg[...] + 1.0)
    z_ref[...] = xv @ w1a[...] + sv @ w1b[...]
    u_ref[...] = sv * dinv


def _tc_pre(xp, sp, wpre, bpre, wes, bes, w1a, w1b, dg):
    row = lambda i: (i, 0)
    whole = lambda i: (0, 0)
    return pl.pallas_call(
        _pre_body,
        grid=(_GRID,),
        in_specs=[
            pl.BlockSpec((_BLK, 128), row),
            pl.BlockSpec((_BLK, 16), row),
            pl.BlockSpec((128, 128), whole),
            pl.BlockSpec((1, 128), whole),
            pl.BlockSpec((16, 128), whole),
            pl.BlockSpec((1, 128), whole),
            pl.BlockSpec((128, 128), whole),
            pl.BlockSpec((128, 128), whole),
            pl.BlockSpec((_BLK, 1), row),
        ],
        out_specs=[
            pl.BlockSpec((_BLK, 128), row),
            pl.BlockSpec((_BLK, 128), row),
        ],
        out_shape=[
            jax.ShapeDtypeStruct((NP, 128), F32),
            jax.ShapeDtypeStruct((NP, 128), F32),
        ],
    )(xp, sp, wpre, bpre, wes, bes, w1a, w1b, dg)


def _layer_update(z, u, a, dinv, b1, w2, b2, bg, wg):
    m = jnp.maximum(z + a[:, :128] + b1, 0.0) @ w2 + b2
    xn = jnp.maximum(m, 0.0)
    sn = jnp.tanh(dinv * ((a[:, 128:] + u) @ wg) + bg)
    return xn, sn


def _lay_body(z_ref, u_ref, a0_ref, a1_ref, dg, b1, w2, b2, bg, wg, w1n,
              zo_ref, uo_ref):
    dinv = lax.rsqrt(dg[...] + 1.0)
    xn, sn = _layer_update(z_ref[...], u_ref[...], a0_ref[...] + a1_ref[...],
                           dinv, b1[...], w2[...], b2[...], bg[...], wg[...])
    zo_ref[...] = jnp.concatenate([xn, sn], axis=1) @ w1n[...]
    uo_ref[...] = sn * dinv


def _tc_layer(z, u, a0, a1, dg, b1, w2, b2, bg, wg, w1n):
    row = lambda i: (i, 0)
    whole = lambda i: (0, 0)
    return pl.pallas_call(
        _lay_body,
        grid=(_GRID,),
        in_specs=[
            pl.BlockSpec((_BLK, 128), row),
            pl.BlockSpec((_BLK, 128), row),
            pl.BlockSpec((_BLK, W), row),
            pl.BlockSpec((_BLK, W), row),
            pl.BlockSpec((_BLK, 1), row),
            pl.BlockSpec((1, 128), whole),
            pl.BlockSpec((128, 128), whole),
            pl.BlockSpec((1, 128), whole),
            pl.BlockSpec((1, 128), whole),
            pl.BlockSpec((128, 128), whole),
            pl.BlockSpec((256, 128), whole),
        ],
        out_specs=[
            pl.BlockSpec((_BLK, 128), row),
            pl.BlockSpec((_BLK, 128), row),
        ],
        out_shape=[
            jax.ShapeDtypeStruct((NP, 128), F32),
            jax.ShapeDtypeStruct((NP, 128), F32),
        ],
    )(z, u, a0, a1, dg, b1, w2, b2, bg, wg, w1n)


def _fin_body(z_ref, u_ref, a0_ref, a1_ref, dg, b1, w2, b2, bg, wg,
              whp, bhp, bat_ref, wpost, bpost, wro, bro,
              out_ref, pool_acc):
    i = pl.program_id(0)
    dinv = lax.rsqrt(dg[...] + 1.0)
    xn, sn = _layer_update(z_ref[...], u_ref[...], a0_ref[...] + a1_ref[...],
                           dinv, b1[...], w2[...], b2[...], bg[...], wg[...])
    hp = jnp.concatenate([xn, sn], axis=1) @ whp[...] + bhp[...]
    onehot = (lax.broadcasted_iota(jnp.int32, (NGR, _BLK), 0)
              == bat_ref[0]).astype(F32)
    part = onehot @ hp

    @pl.when(i == 0)
    def _():
        pool_acc[...] = part

    @pl.when(i > 0)
    def _():
        pool_acc[...] = pool_acc[...] + part

    @pl.when(i == _GRID - 1)
    def _():
        p2 = jnp.maximum(pool_acc[...] @ wpost[...] + bpost[...], 0.0)
        logits = p2 @ wro[...] + bro[...]
        m = jnp.max(logits, axis=1, keepdims=True)
        z = logits - m
        lse = jnp.log(jnp.sum(jnp.exp(z), axis=1, keepdims=True))
        out_ref[...] = z - lse


def _tc_final(z, u, a0, a1, dg, b1, w2, b2, bg, wg,
              whp, bhp, batr, wpost, bpost, wro, bro):
    row = lambda i: (i, 0)
    whole = lambda i: (0, 0)
    return pl.pallas_call(
        _fin_body,
        grid=(_GRID,),
        in_specs=[
            pl.BlockSpec((_BLK, 128), row),
            pl.BlockSpec((_BLK, 128), row),
            pl.BlockSpec((_BLK, W), row),
            pl.BlockSpec((_BLK, W), row),
            pl.BlockSpec((_BLK, 1), row),
            pl.BlockSpec((1, 128), whole),
            pl.BlockSpec((128, 128), whole),
            pl.BlockSpec((1, 128), whole),
            pl.BlockSpec((1, 128), whole),
            pl.BlockSpec((128, 128), whole),
            pl.BlockSpec((256, 128), whole),
            pl.BlockSpec((1, 128), whole),
            pl.BlockSpec((1, 1, _BLK), lambda i: (i, 0, 0)),
            pl.BlockSpec((128, 128), whole),
            pl.BlockSpec((1, 128), whole),
            pl.BlockSpec((128, NCLS), whole),
            pl.BlockSpec((1, NCLS), whole),
        ],
        out_specs=pl.BlockSpec((NGR, NCLS), whole),
        out_shape=jax.ShapeDtypeStruct((NGR, NCLS), F32),
        scratch_shapes=[pltpu.VMEM((NGR, 128), F32)],
    )(z, u, a0, a1, dg, b1, w2, b2, bg, wg,
      whp, bhp, batr, wpost, bpost, wro, bro)


def kernel(x, s, W_pre, b_pre, W_es, b_es, W1s, b1s, W2s, b2s, Wgs, bgs,
           W_hp, b_hp, W_post, b_post, W_ro, b_ro, edge_index, batch):
    src = edge_index[0]
    dst = edge_index[1]
    npad = EP - E
    pad_src = jnp.arange(npad, dtype=jnp.int32) % 2048
    pad_dst = N + jnp.arange(npad, dtype=jnp.int32) % (NP - N)
    src_r = jnp.concatenate([src, pad_src]).reshape(2, NT, NCHUNK, CH)
    dst_r = jnp.concatenate([dst, pad_dst]).reshape(2, NT, NCHUNK, CH)
    zeros_t = jnp.zeros((RPT, 128), F32)

    xp = jnp.pad(x, ((0, NP - N), (0, 0)))
    sp = jnp.pad(s, ((0, NP - N), (0, 0)))
    batr = jnp.pad(batch, (0, NP - N),
                   constant_values=NGR).reshape(_GRID, 1, _BLK)

    degp = _sc_deg(dst_r).reshape(2, NP)
    dg = (degp[0] + degp[1])[:, None]

    r1 = lambda a: a.reshape(1, -1)
    zc, uc = _tc_pre(xp, sp, W_pre, r1(b_pre), W_es, r1(b_es),
                     W1s[0, :128], W1s[0, 128:], dg)
    for i in range(3):
        aggp = _sc_agg(zc, uc, src_r, dst_r, zeros_t)
        if i < 2:
            zc, uc = _tc_layer(
                zc, uc, aggp[0], aggp[1], dg,
                r1(b1s[i]), W2s[i], r1(b2s[i]), r1(bgs[i]), Wgs[i],
                W1s[i + 1])
        else:
            out = _tc_final(
                zc, uc, aggp[0], aggp[1], dg,
                r1(b1s[i]), W2s[i], r1(b2s[i]), r1(bgs[i]), Wgs[i],
                W_hp, r1(b_hp), batr, W_post, r1(b_post), W_ro, r1(b_ro))
    return out


# speedup vs baseline: 13.4537x; 1.1644x over previous
import jax
import jax.numpy as jnp
from jax import lax
from jax.experimental import pallas as pl
from jax.experimental.pallas import tpu as pltpu
from jax.experimental.pallas import tpu_sc as plsc

N = 10000
NP = 10240
E = 320000
NT = 16
CH = 64
NCHUNK = 160
NSEG = 4
NBUF = 4
EP = 2 * NT * NCHUNK * CH
RPT = NP // NT
W = 256
NGR = 128
NCLS = 10
F32 = jnp.float32

_MESH = dict(core_axis_name="c", subcore_axis_name="s", num_cores=2,
             num_subcores=16)


def _deg_body(dst_hbm, out_hbm, dst_v, ones_v, zbuf, acc1):
    c = lax.axis_index("c")
    sid = lax.axis_index("s")
    pltpu.sync_copy(dst_hbm.at[c, sid], dst_v)
    for i in range(CH // 16):
        ones_v[pl.ds(i * 16, 16)] = jnp.full((16,), 1.0, F32)
    for i in range(RPT // 16):
        zbuf[pl.ds(i * 16, 16)] = jnp.zeros((16,), F32)
    pltpu.sync_copy(zbuf, acc1.at[pl.ds(sid * RPT, RPT)])
    plsc.subcore_barrier()

    def chunk(j, carry):
        pltpu.sync_copy(ones_v, acc1.at[dst_v.at[j]], add=True)
        return carry

    lax.fori_loop(0, NCHUNK, chunk, 0)
    plsc.subcore_barrier()
    off = pl.multiple_of(c * NP + sid * RPT, 128)
    pltpu.sync_copy(acc1.at[pl.ds(sid * RPT, RPT)],
                    out_hbm.at[pl.ds(off, RPT)])


def _sc_deg(dst_r):
    k = pl.kernel(
        _deg_body,
        out_type=jax.ShapeDtypeStruct((2 * NP,), F32),
        mesh=plsc.VectorSubcoreMesh(**_MESH),
        scratch_types=[
            pltpu.VMEM((NCHUNK, CH), jnp.int32),
            pltpu.VMEM((CH,), F32),
            pltpu.VMEM((RPT,), F32),
            pltpu.VMEM_SHARED((NP,), F32),
        ],
    )
    return k(dst_r)


HC = NCHUNK // NSEG


def _agg_body(t0_hbm, t1_hbm, src_hbm, dst_hbm, z_hbm, out_hbm,
              src_v, dst_v, b0, b1, b2, b3,
              g0, g1, g2, g3, s0, s1, s2, s3, acc):
    c = lax.axis_index("c")
    sid = lax.axis_index("s")
    bufs = (b0, b1, b2, b3)
    gsems = (g0, g1, g2, g3)
    ssems = (s0, s1, s2, s3)
    for k, tab in enumerate((t0_hbm, t1_hbm)):
        pltpu.sync_copy(z_hbm, acc.at[pl.ds(sid * RPT, RPT)])
        plsc.subcore_barrier()

        def wait_rows(buf, sem_, tab=tab):
            pltpu.make_async_copy(tab.at[src_v.at[0]], buf, sem_).wait()

        def wait_scat(buf, sem_):
            pltpu.make_async_copy(buf, acc.at[dst_v.at[0]], sem_).wait()

        for half in range(NSEG):
            pltpu.sync_copy(src_hbm.at[c, sid, pl.ds(half * HC, HC)], src_v)
            pltpu.sync_copy(dst_hbm.at[c, sid, pl.ds(half * HC, HC)], dst_v)

            for l in range(NBUF):
                pltpu.async_copy(tab.at[src_v.at[l]], bufs[l], gsems[l])

            def group(g, carry, tab=tab):
                j = g * NBUF
                for l in range(NBUF):
                    wait_rows(bufs[l], gsems[l])
                    pltpu.async_copy(bufs[l], acc.at[dst_v.at[j + l]],
                                     ssems[l], add=True)
                for l in range(NBUF):
                    wait_scat(bufs[l], ssems[l])

                    @pl.when(j + l + NBUF < HC)
                    def _(l=l):
                        pltpu.async_copy(tab.at[src_v.at[j + l + NBUF]],
                                         bufs[l], gsems[l])

                return carry

            lax.fori_loop(0, HC // NBUF, group, 0)
        plsc.subcore_barrier()
        pltpu.sync_copy(acc.at[pl.ds(sid * RPT, RPT)],
                        out_hbm.at[c, pl.ds(sid * RPT, RPT),
                                   pl.ds(k * 128, 128)])


def _sc_agg(t0, t1, src_r, dst_r, zeros_t):
    k = pl.kernel(
        _agg_body,
        out_type=jax.ShapeDtypeStruct((2, NP, W), F32),
        mesh=plsc.VectorSubcoreMesh(**_MESH),
        scratch_types=(
            [pltpu.VMEM((HC, CH), jnp.int32)] * 2
            + [pltpu.VMEM((CH, 128), F32)] * 4
            + [pltpu.SemaphoreType.DMA] * 8
            + [pltpu.VMEM_SHARED((NP, 128), F32)]
        ),
    )
    return k(t0, t1, src_r, dst_r, zeros_t)


_BLK = 1280
_GRID = NP // _BLK


def _pre_body(xp_ref, sp_ref, wpre, bpre, wes, bes, w1a, w1b, dg,
              z_ref, u_ref):
    xv = xp_ref[...] @ wpre[...] + bpre[...]
    sv = sp_ref[...] @ wes[...] + bes[...]
    dinv = lax.rsqrt(dg[...] + 1.0)
    z_ref[...] = xv @ w1a[...] + sv @ w1b[...]
    u_ref[...] = sv * dinv


def _tc_pre(xp, sp, wpre, bpre, wes, bes, w1a, w1b, dg):
    row = lambda i: (i, 0)
    whole = lambda i: (0, 0)
    return pl.pallas_call(
        _pre_body,
        grid=(_GRID,),
        in_specs=[
            pl.BlockSpec((_BLK, 128), row),
            pl.BlockSpec((_BLK, 16), row),
            pl.BlockSpec((128, 128), whole),
            pl.BlockSpec((1, 128), whole),
            pl.BlockSpec((16, 128), whole),
            pl.BlockSpec((1, 128), whole),
            pl.BlockSpec((128, 128), whole),
            pl.BlockSpec((128, 128), whole),
            pl.BlockSpec((_BLK, 1), row),
        ],
        out_specs=[
            pl.BlockSpec((_BLK, 128), row),
            pl.BlockSpec((_BLK, 128), row),
        ],
        out_shape=[
            jax.ShapeDtypeStruct((NP, 128), F32),
            jax.ShapeDtypeStruct((NP, 128), F32),
        ],
    )(xp, sp, wpre, bpre, wes, bes, w1a, w1b, dg)


def _layer_update(z, u, a, dinv, b1, w2, b2, bg, wg):
    m = jnp.maximum(z + a[:, :128] + b1, 0.0) @ w2 + b2
    xn = jnp.maximum(m, 0.0)
    sn = jnp.tanh(dinv * ((a[:, 128:] + u) @ wg) + bg)
    return xn, sn


def _lay_body(z_ref, u_ref, a0_ref, a1_ref, dg, b1, w2, b2, bg, wg, w1n,
              zo_ref, uo_ref):
    dinv = lax.rsqrt(dg[...] + 1.0)
    xn, sn = _layer_update(z_ref[...], u_ref[...], a0_ref[...] + a1_ref[...],
                           dinv, b1[...], w2[...], b2[...], bg[...], wg[...])
    zo_ref[...] = jnp.concatenate([xn, sn], axis=1) @ w1n[...]
    uo_ref[...] = sn * dinv


def _tc_layer(z, u, a0, a1, dg, b1, w2, b2, bg, wg, w1n):
    row = lambda i: (i, 0)
    whole = lambda i: (0, 0)
    return pl.pallas_call(
        _lay_body,
        grid=(_GRID,),
        in_specs=[
            pl.BlockSpec((_BLK, 128), row),
            pl.BlockSpec((_BLK, 128), row),
            pl.BlockSpec((_BLK, W), row),
            pl.BlockSpec((_BLK, W), row),
            pl.BlockSpec((_BLK, 1), row),
            pl.BlockSpec((1, 128), whole),
            pl.BlockSpec((128, 128), whole),
            pl.BlockSpec((1, 128), whole),
            pl.BlockSpec((1, 128), whole),
            pl.BlockSpec((128, 128), whole),
            pl.BlockSpec((256, 128), whole),
        ],
        out_specs=[
            pl.BlockSpec((_BLK, 128), row),
            pl.BlockSpec((_BLK, 128), row),
        ],
        out_shape=[
            jax.ShapeDtypeStruct((NP, 128), F32),
            jax.ShapeDtypeStruct((NP, 128), F32),
        ],
    )(z, u, a0, a1, dg, b1, w2, b2, bg, wg, w1n)


def _fin_body(z_ref, u_ref, a0_ref, a1_ref, dg, b1, w2, b2, bg, wg,
              whp, bhp, bat_ref, wpost, bpost, wro, bro,
              out_ref, pool_acc):
    i = pl.program_id(0)
    dinv = lax.rsqrt(dg[...] + 1.0)
    xn, sn = _layer_update(z_ref[...], u_ref[...], a0_ref[...] + a1_ref[...],
                           dinv, b1[...], w2[...], b2[...], bg[...], wg[...])
    hp = jnp.concatenate([xn, sn], axis=1) @ whp[...] + bhp[...]
    onehot = (lax.broadcasted_iota(jnp.int32, (NGR, _BLK), 0)
              == bat_ref[0]).astype(F32)
    part = onehot @ hp

    @pl.when(i == 0)
    def _():
        pool_acc[...] = part

    @pl.when(i > 0)
    def _():
        pool_acc[...] = pool_acc[...] + part

    @pl.when(i == _GRID - 1)
    def _():
        p2 = jnp.maximum(pool_acc[...] @ wpost[...] + bpost[...], 0.0)
        logits = p2 @ wro[...] + bro[...]
        m = jnp.max(logits, axis=1, keepdims=True)
        z = logits - m
        lse = jnp.log(jnp.sum(jnp.exp(z), axis=1, keepdims=True))
        out_ref[...] = z - lse


def _tc_final(z, u, a0, a1, dg, b1, w2, b2, bg, wg,
              whp, bhp, batr, wpost, bpost, wro, bro):
    row = lambda i: (i, 0)
    whole = lambda i: (0, 0)
    return pl.pallas_call(
        _fin_body,
        grid=(_GRID,),
        in_specs=[
            pl.BlockSpec((_BLK, 128), row),
            pl.BlockSpec((_BLK, 128), row),
            pl.BlockSpec((_BLK, W), row),
            pl.BlockSpec((_BLK, W), row),
            pl.BlockSpec((_BLK, 1), row),
            pl.BlockSpec((1, 128), whole),
            pl.BlockSpec((128, 128), whole),
            pl.BlockSpec((1, 128), whole),
            pl.BlockSpec((1, 128), whole),
            pl.BlockSpec((128, 128), whole),
            pl.BlockSpec((256, 128), whole),
            pl.BlockSpec((1, 128), whole),
            pl.BlockSpec((1, 1, _BLK), lambda i: (i, 0, 0)),
            pl.BlockSpec((128, 128), whole),
            pl.BlockSpec((1, 128), whole),
            pl.BlockSpec((128, NCLS), whole),
            pl.BlockSpec((1, NCLS), whole),
        ],
        out_specs=pl.BlockSpec((NGR, NCLS), whole),
        out_shape=jax.ShapeDtypeStruct((NGR, NCLS), F32),
        scratch_shapes=[pltpu.VMEM((NGR, 128), F32)],
    )(z, u, a0, a1, dg, b1, w2, b2, bg, wg,
      whp, bhp, batr, wpost, bpost, wro, bro)


def kernel(x, s, W_pre, b_pre, W_es, b_es, W1s, b1s, W2s, b2s, Wgs, bgs,
           W_hp, b_hp, W_post, b_post, W_ro, b_ro, edge_index, batch):
    src = edge_index[0]
    dst = edge_index[1]
    npad = EP - E
    pad_src = jnp.arange(npad, dtype=jnp.int32) % 2048
    pad_dst = N + jnp.arange(npad, dtype=jnp.int32) % (NP - N)
    src_r = jnp.concatenate([src, pad_src]).reshape(2, NT, NCHUNK, CH)
    dst_r = jnp.concatenate([dst, pad_dst]).reshape(2, NT, NCHUNK, CH)
    zeros_t = jnp.zeros((RPT, 128), F32)

    xp = jnp.pad(x, ((0, NP - N), (0, 0)))
    sp = jnp.pad(s, ((0, NP - N), (0, 0)))
    batr = jnp.pad(batch, (0, NP - N),
                   constant_values=NGR).reshape(_GRID, 1, _BLK)

    degp = _sc_deg(dst_r).reshape(2, NP)
    dg = (degp[0] + degp[1])[:, None]

    r1 = lambda a: a.reshape(1, -1)
    zc, uc = _tc_pre(xp, sp, W_pre, r1(b_pre), W_es, r1(b_es),
                     W1s[0, :128], W1s[0, 128:], dg)
    for i in range(3):
        aggp = _sc_agg(zc, uc, src_r, dst_r, zeros_t)
        if i < 2:
            zc, uc = _tc_layer(
                zc, uc, aggp[0], aggp[1], dg,
                r1(b1s[i]), W2s[i], r1(b2s[i]), r1(bgs[i]), Wgs[i],
                W1s[i + 1])
        else:
            out = _tc_final(
                zc, uc, aggp[0], aggp[1], dg,
                r1(b1s[i]), W2s[i], r1(b2s[i]), r1(bgs[i]), Wgs[i],
                W_hp, r1(b_hp), batr, W_post, r1(b_post), W_ro, r1(b_ro))
    return out


# speedup vs baseline: 13.5023x; 1.0036x over previous
import jax
import jax.numpy as jnp
from jax import lax
from jax.experimental import pallas as pl
from jax.experimental.pallas import tpu as pltpu
from jax.experimental.pallas import tpu_sc as plsc

N = 10000
NP = 10240
E = 320000
NT = 16
CH = 64
NCHUNK = 160
NSEG = 4
NBUF = 4
EP = 2 * NT * NCHUNK * CH
RPT = NP // NT
W = 256
NGR = 128
NCLS = 10
F32 = jnp.float32

_MESH = dict(core_axis_name="c", subcore_axis_name="s", num_cores=2,
             num_subcores=16)


def _deg_body(dst_hbm, out_hbm, dst_v, ones_v, zbuf, acc1):
    c = lax.axis_index("c")
    sid = lax.axis_index("s")
    pltpu.sync_copy(dst_hbm.at[c, sid], dst_v)
    for i in range(CH // 16):
        ones_v[pl.ds(i * 16, 16)] = jnp.full((16,), 1.0, F32)
    for i in range(RPT // 16):
        zbuf[pl.ds(i * 16, 16)] = jnp.zeros((16,), F32)
    pltpu.sync_copy(zbuf, acc1.at[pl.ds(sid * RPT, RPT)])
    plsc.subcore_barrier()

    def chunk(j, carry):
        pltpu.sync_copy(ones_v, acc1.at[dst_v.at[j]], add=True)
        return carry

    lax.fori_loop(0, NCHUNK, chunk, 0)
    plsc.subcore_barrier()
    off = pl.multiple_of(c * NP + sid * RPT, 128)
    pltpu.sync_copy(acc1.at[pl.ds(sid * RPT, RPT)],
                    out_hbm.at[pl.ds(off, RPT)])


def _sc_deg(dst_r):
    k = pl.kernel(
        _deg_body,
        out_type=jax.ShapeDtypeStruct((2 * NP,), F32),
        mesh=plsc.VectorSubcoreMesh(**_MESH),
        scratch_types=[
            pltpu.VMEM((NCHUNK, CH), jnp.int32),
            pltpu.VMEM((CH,), F32),
            pltpu.VMEM((RPT,), F32),
            pltpu.VMEM_SHARED((NP,), F32),
        ],
    )
    return k(dst_r)


HC = NCHUNK // NSEG


def _agg_body(t0_hbm, t1_hbm, src_hbm, dst_hbm, z_hbm, out_hbm,
              src_v, dst_v, b0, b1, b2, b3,
              g0, g1, g2, g3, s0, s1, s2, s3, acc):
    c = lax.axis_index("c")
    sid = lax.axis_index("s")
    bufs = (b0, b1, b2, b3)
    gsems = (g0, g1, g2, g3)
    ssems = (s0, s1, s2, s3)
    for k, tab in enumerate((t0_hbm, t1_hbm)):
        pltpu.sync_copy(z_hbm, acc.at[pl.ds(sid * RPT, RPT)])
        plsc.subcore_barrier()

        def wait_rows(buf, sem_, tab=tab):
            pltpu.make_async_copy(tab.at[src_v.at[0]], buf, sem_).wait()

        def wait_scat(buf, sem_):
            pltpu.make_async_copy(buf, acc.at[dst_v.at[0]], sem_).wait()

        for half in range(NSEG):
            pltpu.sync_copy(src_hbm.at[c, sid, pl.ds(half * HC, HC)], src_v)
            pltpu.sync_copy(dst_hbm.at[c, sid, pl.ds(half * HC, HC)], dst_v)

            for l in range(NBUF):
                pltpu.async_copy(tab.at[src_v.at[l]], bufs[l], gsems[l])

            def group(g, carry, tab=tab):
                j = g * NBUF
                for l in range(NBUF):
                    wait_rows(bufs[l], gsems[l])
                    pltpu.async_copy(bufs[l], acc.at[dst_v.at[j + l]],
                                     ssems[l], add=True)
                for l in range(NBUF):
                    wait_scat(bufs[l], ssems[l])

                    @pl.when(j + l + NBUF < HC)
                    def _(l=l):
                        pltpu.async_copy(tab.at[src_v.at[j + l + NBUF]],
                                         bufs[l], gsems[l])

                return carry

            lax.fori_loop(0, HC // NBUF, group, 0)
        plsc.subcore_barrier()
        pltpu.sync_copy(acc.at[pl.ds(sid * RPT, RPT)],
                        out_hbm.at[c, pl.ds(sid * RPT, RPT),
                                   pl.ds(k * 128, 128)])


def _sc_agg(t0, t1, src_r, dst_r, zeros_t):
    k = pl.kernel(
        _agg_body,
        out_type=jax.ShapeDtypeStruct((2, NP, W), F32),
        mesh=plsc.VectorSubcoreMesh(**_MESH),
        scratch_types=(
            [pltpu.VMEM((HC, CH), jnp.int32)] * 2
            + [pltpu.VMEM((CH, 128), F32)] * 4
            + [pltpu.SemaphoreType.DMA] * 8
            + [pltpu.VMEM_SHARED((NP, 128), F32)]
        ),
    )
    return k(t0, t1, src_r, dst_r, zeros_t)


_BLK = 2560
_GRID = NP // _BLK


def _pre_body(xp_ref, sp_ref, wpre, bpre, wes, bes, w1a, w1b, dg,
              z_ref, u_ref):
    xv = xp_ref[...] @ wpre[...] + bpre[...]
    sv = sp_ref[...] @ wes[...] + bes[...]
    dinv = lax.rsqrt(dg[...] + 1.0)
    z_ref[...] = xv @ w1a[...] + sv @ w1b[...]
    u_ref[...] = sv * dinv


def _tc_pre(xp, sp, wpre, bpre, wes, bes, w1a, w1b, dg):
    row = lambda i: (i, 0)
    whole = lambda i: (0, 0)
    return pl.pallas_call(
        _pre_body,
        grid=(_GRID,),
        in_specs=[
            pl.BlockSpec((_BLK, 128), row),
            pl.BlockSpec((_BLK, 16), row),
            pl.BlockSpec((128, 128), whole),
            pl.BlockSpec((1, 128), whole),
            pl.BlockSpec((16, 128), whole),
            pl.BlockSpec((1, 128), whole),
            pl.BlockSpec((128, 128), whole),
            pl.BlockSpec((128, 128), whole),
            pl.BlockSpec((_BLK, 1), row),
        ],
        out_specs=[
            pl.BlockSpec((_BLK, 128), row),
            pl.BlockSpec((_BLK, 128), row),
        ],
        out_shape=[
            jax.ShapeDtypeStruct((NP, 128), F32),
            jax.ShapeDtypeStruct((NP, 128), F32),
        ],
    )(xp, sp, wpre, bpre, wes, bes, w1a, w1b, dg)


def _layer_update(z, u, a, dinv, b1, w2, b2, bg, wg):
    m = jnp.maximum(z + a[:, :128] + b1, 0.0) @ w2 + b2
    xn = jnp.maximum(m, 0.0)
    sn = jnp.tanh(dinv * ((a[:, 128:] + u) @ wg) + bg)
    return xn, sn


def _lay_body(z_ref, u_ref, a0_ref, a1_ref, dg, b1, w2, b2, bg, wg, w1n,
              zo_ref, uo_ref):
    dinv = lax.rsqrt(dg[...] + 1.0)
    xn, sn = _layer_update(z_ref[...], u_ref[...], a0_ref[...] + a1_ref[...],
                           dinv, b1[...], w2[...], b2[...], bg[...], wg[...])
    zo_ref[...] = jnp.concatenate([xn, sn], axis=1) @ w1n[...]
    uo_ref[...] = sn * dinv


def _tc_layer(z, u, a0, a1, dg, b1, w2, b2, bg, wg, w1n):
    row = lambda i: (i, 0)
    whole = lambda i: (0, 0)
    return pl.pallas_call(
        _lay_body,
        grid=(_GRID,),
        in_specs=[
            pl.BlockSpec((_BLK, 128), row),
            pl.BlockSpec((_BLK, 128), row),
            pl.BlockSpec((_BLK, W), row),
            pl.BlockSpec((_BLK, W), row),
            pl.BlockSpec((_BLK, 1), row),
            pl.BlockSpec((1, 128), whole),
            pl.BlockSpec((128, 128), whole),
            pl.BlockSpec((1, 128), whole),
            pl.BlockSpec((1, 128), whole),
            pl.BlockSpec((128, 128), whole),
            pl.BlockSpec((256, 128), whole),
        ],
        out_specs=[
            pl.BlockSpec((_BLK, 128), row),
            pl.BlockSpec((_BLK, 128), row),
        ],
        out_shape=[
            jax.ShapeDtypeStruct((NP, 128), F32),
            jax.ShapeDtypeStruct((NP, 128), F32),
        ],
    )(z, u, a0, a1, dg, b1, w2, b2, bg, wg, w1n)


def _fin_body(z_ref, u_ref, a0_ref, a1_ref, dg, b1, w2, b2, bg, wg,
              whp, bhp, bat_ref, wpost, bpost, wro, bro,
              out_ref, pool_acc):
    i = pl.program_id(0)
    dinv = lax.rsqrt(dg[...] + 1.0)
    xn, sn = _layer_update(z_ref[...], u_ref[...], a0_ref[...] + a1_ref[...],
                           dinv, b1[...], w2[...], b2[...], bg[...], wg[...])
    hp = jnp.concatenate([xn, sn], axis=1) @ whp[...] + bhp[...]
    onehot = (lax.broadcasted_iota(jnp.int32, (NGR, _BLK), 0)
              == bat_ref[0]).astype(F32)
    part = onehot @ hp

    @pl.when(i == 0)
    def _():
        pool_acc[...] = part

    @pl.when(i > 0)
    def _():
        pool_acc[...] = pool_acc[...] + part

    @pl.when(i == _GRID - 1)
    def _():
        p2 = jnp.maximum(pool_acc[...] @ wpost[...] + bpost[...], 0.0)
        logits = p2 @ wro[...] + bro[...]
        m = jnp.max(logits, axis=1, keepdims=True)
        z = logits - m
        lse = jnp.log(jnp.sum(jnp.exp(z), axis=1, keepdims=True))
        out_ref[...] = z - lse


def _tc_final(z, u, a0, a1, dg, b1, w2, b2, bg, wg,
              whp, bhp, batr, wpost, bpost, wro, bro):
    row = lambda i: (i, 0)
    whole = lambda i: (0, 0)
    return pl.pallas_call(
        _fin_body,
        grid=(_GRID,),
        in_specs=[
            pl.BlockSpec((_BLK, 128), row),
            pl.BlockSpec((_BLK, 128), row),
            pl.BlockSpec((_BLK, W), row),
            pl.BlockSpec((_BLK, W), row),
            pl.BlockSpec((_BLK, 1), row),
            pl.BlockSpec((1, 128), whole),
            pl.BlockSpec((128, 128), whole),
            pl.BlockSpec((1, 128), whole),
            pl.BlockSpec((1, 128), whole),
            pl.BlockSpec((128, 128), whole),
            pl.BlockSpec((256, 128), whole),
            pl.BlockSpec((1, 128), whole),
            pl.BlockSpec((1, 1, _BLK), lambda i: (i, 0, 0)),
            pl.BlockSpec((128, 128), whole),
            pl.BlockSpec((1, 128), whole),
            pl.BlockSpec((128, NCLS), whole),
            pl.BlockSpec((1, NCLS), whole),
        ],
        out_specs=pl.BlockSpec((NGR, NCLS), whole),
        out_shape=jax.ShapeDtypeStruct((NGR, NCLS), F32),
        scratch_shapes=[pltpu.VMEM((NGR, 128), F32)],
    )(z, u, a0, a1, dg, b1, w2, b2, bg, wg,
      whp, bhp, batr, wpost, bpost, wro, bro)


def kernel(x, s, W_pre, b_pre, W_es, b_es, W1s, b1s, W2s, b2s, Wgs, bgs,
           W_hp, b_hp, W_post, b_post, W_ro, b_ro, edge_index, batch):
    src = edge_index[0]
    dst = edge_index[1]
    npad = EP - E
    pad_src = jnp.arange(npad, dtype=jnp.int32) % 2048
    pad_dst = N + jnp.arange(npad, dtype=jnp.int32) % (NP - N)
    src_r = jnp.concatenate([src, pad_src]).reshape(2, NT, NCHUNK, CH)
    dst_r = jnp.concatenate([dst, pad_dst]).reshape(2, NT, NCHUNK, CH)
    zeros_t = jnp.zeros((RPT, 128), F32)

    xp = jnp.pad(x, ((0, NP - N), (0, 0)))
    sp = jnp.pad(s, ((0, NP - N), (0, 0)))
    batr = jnp.pad(batch, (0, NP - N),
                   constant_values=NGR).reshape(_GRID, 1, _BLK)

    degp = _sc_deg(dst_r).reshape(2, NP)
    dg = (degp[0] + degp[1])[:, None]

    r1 = lambda a: a.reshape(1, -1)
    zc, uc = _tc_pre(xp, sp, W_pre, r1(b_pre), W_es, r1(b_es),
                     W1s[0, :128], W1s[0, 128:], dg)
    for i in range(3):
        aggp = _sc_agg(zc, uc, src_r, dst_r, zeros_t)
        if i < 2:
            zc, uc = _tc_layer(
                zc, uc, aggp[0], aggp[1], dg,
                r1(b1s[i]), W2s[i], r1(b2s[i]), r1(bgs[i]), Wgs[i],
                W1s[i + 1])
        else:
            out = _tc_final(
                zc, uc, aggp[0], aggp[1], dg,
                r1(b1s[i]), W2s[i], r1(b2s[i]), r1(bgs[i]), Wgs[i],
                W_hp, r1(b_hp), batr, W_post, r1(b_post), W_ro, r1(b_ro))
    return out
